# baseline (speedup 1.0000x reference)
import functools

import jax
import jax.numpy as jnp
from jax.experimental import pallas as pl
from jax.experimental.pallas import tpu as pltpu


def _ceil_to(x, m):
    return (x + m - 1) // m * m


def _tile(dim, cap, align):
    if dim <= cap:
        return dim
    t = (cap // align) * align
    while t > align and dim % t:
        t -= align
    assert dim % t == 0, (dim, cap, align)
    return t


def _div_tile(dim, cap):
    for t in range(min(dim, cap), 0, -1):
        if dim % t == 0:
            return t
    return 1


def _permute_gates(w, H):
    return jnp.concatenate([w[..., :2 * H], w[..., 3 * H:], w[..., 2 * H:3 * H]],
                           axis=-1)


def _gates_body(*refs, n_in):
    a_refs = refs[:n_in]
    w_refs = refs[n_in:2 * n_in]
    b_ref = refs[2 * n_in]
    o_ref = refs[2 * n_in + 1]
    acc = jnp.dot(a_refs[0][...].astype(jnp.bfloat16), w_refs[0][...],
                  preferred_element_type=jnp.float32)
    for i in range(1, n_in):
        acc = acc + jnp.dot(a_refs[i][...].astype(jnp.bfloat16), w_refs[i][...],
                            preferred_element_type=jnp.float32)
    o_ref[...] = (acc + b_ref[...]).astype(o_ref.dtype)


def _input_gates(a_list, w_list, b, *, tm_cap=512, tn_cap=512):
    n_in = len(a_list)
    M = a_list[0].shape[0]
    G, _, N = w_list[0].shape
    tm = _tile(M, tm_cap, 8)
    tn = _tile(N, tn_cap, 128)
    grid = (G, N // tn, M // tm)
    in_specs = []
    for a in a_list:
        in_specs.append(pl.BlockSpec((tm, a.shape[1]), lambda g, n, m: (m, 0)))
    for w in w_list:
        in_specs.append(pl.BlockSpec((None, w.shape[1], tn),
                                     lambda g, n, m: (g, 0, n)))
    in_specs.append(pl.BlockSpec((None, 1, tn), lambda g, n, m: (g, 0, n)))
    tile_bytes = (sum(2 * tm * a.shape[1] * a.dtype.itemsize for a in a_list)
                  + sum(2 * w.shape[1] * tn * 2 for w in w_list)
                  + 2 * tn * 4 + 2 * tm * tn * 2)
    vmem_limit = int(min(64 * 1024 * 1024, max(16 * 1024 * 1024, 2 * tile_bytes)))
    return pl.pallas_call(
        functools.partial(_gates_body, n_in=n_in),
        out_shape=jax.ShapeDtypeStruct((G, M, N), jnp.bfloat16),
        grid=grid,
        in_specs=in_specs,
        out_specs=pl.BlockSpec((None, tm, tn), lambda g, n, m: (g, m, n)),
        compiler_params=pltpu.CompilerParams(
            dimension_semantics=("parallel", "parallel", "parallel"),
            vmem_limit_bytes=vmem_limit),
    )(*a_list, *w_list, b)


def _lstm_body(gf_ref, gb_ref, whh_ref, hf_ref, hb_ref,
               hf_sc, cf_sc, hb_sc, cb_sc, *, H, tc):
    @pl.when(pl.program_id(1) == 0)
    def _():
        hf_sc[...] = jnp.zeros_like(hf_sc)
        cf_sc[...] = jnp.zeros_like(cf_sc)
        hb_sc[...] = jnp.zeros_like(hb_sc)
        cb_sc[...] = jnp.zeros_like(cb_sc)

    wf = whh_ref[0]
    wb = whh_ref[1]
    hf, cf = hf_sc[...], cf_sc[...]
    hb, cb = hb_sc[...], cb_sc[...]
    bf16 = jnp.bfloat16
    for t in range(tc):
        tb = tc - 1 - t
        zf = gf_ref[t].astype(jnp.float32) + jnp.dot(
            hf.astype(bf16), wf, preferred_element_type=jnp.float32)
        zb = gb_ref[tb].astype(jnp.float32) + jnp.dot(
            hb.astype(bf16), wb, preferred_element_type=jnp.float32)
        pf = jax.nn.sigmoid(zf[:, :3 * H])
        pb = jax.nn.sigmoid(zb[:, :3 * H])
        cf = pf[:, H:2 * H] * cf + pf[:, :H] * jnp.tanh(zf[:, 3 * H:])
        cb = pb[:, H:2 * H] * cb + pb[:, :H] * jnp.tanh(zb[:, 3 * H:])
        hf = pf[:, 2 * H:] * jnp.tanh(cf)
        hb = pb[:, 2 * H:] * jnp.tanh(cb)
        hf_ref[t] = hf.astype(bf16)
        hb_ref[tb] = hb.astype(bf16)
    hf_sc[...], cf_sc[...] = hf, cf
    hb_sc[...], cb_sc[...] = hb, cb


def _bilstm(g, whh, *, tc_cap=16):
    _, T, Bp, H4 = g.shape
    H = H4 // 4
    tc = _div_tile(T, tc_cap)
    nc = T // tc
    nb = 2 if Bp % 16 == 0 else 1
    Bh = Bp // nb

    out_shape = [jax.ShapeDtypeStruct((T, Bp, H), jnp.bfloat16)] * 2
    return pl.pallas_call(
        functools.partial(_lstm_body, H=H, tc=tc),
        out_shape=out_shape,
        grid=(nb, nc),
        in_specs=[
            pl.BlockSpec((None, tc, Bh, H4), lambda b, c: (0, c, b, 0)),
            pl.BlockSpec((None, tc, Bh, H4),
                         lambda b, c, nc=nc: (1, nc - 1 - c, b, 0)),
            pl.BlockSpec((2, H, H4), lambda b, c: (0, 0, 0)),
        ],
        out_specs=[
            pl.BlockSpec((tc, Bh, H), lambda b, c: (c, b, 0)),
            pl.BlockSpec((tc, Bh, H), lambda b, c, nc=nc: (nc - 1 - c, b, 0)),
        ],
        scratch_shapes=[
            pltpu.VMEM((Bh, H), jnp.float32),
            pltpu.VMEM((Bh, H), jnp.float32),
            pltpu.VMEM((Bh, H), jnp.float32),
            pltpu.VMEM((Bh, H), jnp.float32),
        ],
        compiler_params=pltpu.CompilerParams(
            dimension_semantics=("parallel", "arbitrary")),
    )(g, g, whh)


def _head_body(h_ref, w_ref, b_ref, o_ref, *, H, tt):
    a0 = h_ref[0].reshape(-1, H)
    a1 = h_ref[1].reshape(-1, H)
    acc = jnp.dot(a0, w_ref[0], preferred_element_type=jnp.float32)
    acc = acc + jnp.dot(a1, w_ref[1], preferred_element_type=jnp.float32)
    y = jnp.tanh(acc + b_ref[...])
    y = y.reshape(tt, -1, y.shape[-1])
    for i in range(tt):
        o_ref[:, i, :] = y[i]


def _head(h, w, b, *, tt_cap=8, tn_cap=512):
    _, T, Bp, H = h.shape
    N = w.shape[-1]
    tn = _tile(N, tn_cap, 128)
    tt = _div_tile(T, tt_cap)
    grid = (N // tn, T // tt)
    return pl.pallas_call(
        functools.partial(_head_body, H=H, tt=tt),
        out_shape=jax.ShapeDtypeStruct((Bp, T, N), jnp.float32),
        grid=grid,
        in_specs=[
            pl.BlockSpec((2, tt, Bp, H), lambda n, t: (0, t, 0, 0)),
            pl.BlockSpec((2, H, tn), lambda n, t: (0, 0, n)),
            pl.BlockSpec((1, tn), lambda n, t: (0, n)),
        ],
        out_specs=pl.BlockSpec((Bp, tt, tn), lambda n, t: (0, t, n)),
        compiler_params=pltpu.CompilerParams(
            dimension_semantics=("parallel", "parallel")),
    )(h, w, b)


def _head_bm_body(a0_ref, a1_ref, w0_ref, w1_ref, b_ref, o_ref):
    acc = jnp.dot(a0_ref[...], w0_ref[...], preferred_element_type=jnp.float32)
    acc = acc + jnp.dot(a1_ref[...], w1_ref[...], preferred_element_type=jnp.float32)
    o_ref[...] = jnp.tanh(acc + b_ref[...])


def _head_bm(hf, hb, w0, w1, b, *, tn_cap=512):
    T, Bp, H = hf.shape
    N = w0.shape[-1]
    tn = _tile(N, tn_cap, 128)
    a0 = hf.reshape(T, Bp * H)
    a1 = hb.reshape(T, Bp * H)
    grid = (N // tn, Bp)
    tile_bytes = (2 * 2 * T * H * 2 + 2 * 2 * H * tn * 2 + 2 * tn * 4
                  + 2 * T * tn * 4)
    vmem_limit = int(min(64 * 1024 * 1024, max(16 * 1024 * 1024, 2 * tile_bytes)))
    return pl.pallas_call(
        _head_bm_body,
        out_shape=jax.ShapeDtypeStruct((Bp, T, N), jnp.float32),
        grid=grid,
        in_specs=[
            pl.BlockSpec((T, H), lambda n, b: (0, b)),
            pl.BlockSpec((T, H), lambda n, b: (0, b)),
            pl.BlockSpec((None, H, tn), lambda n, b: (0, 0, n)),
            pl.BlockSpec((None, H, tn), lambda n, b: (0, 0, n)),
            pl.BlockSpec((None, 1, tn), lambda n, b: (0, 0, n)),
        ],
        out_specs=pl.BlockSpec((None, T, tn), lambda n, b: (b, 0, n)),
        compiler_params=pltpu.CompilerParams(
            dimension_semantics=("parallel", "parallel"),
            vmem_limit_bytes=vmem_limit),
    )(a0, a1, w0, w1, b)


def _head_tm_body(a0_ref, a1_ref, w0_ref, w1_ref, b_ref, o_ref):
    acc = jnp.dot(a0_ref[...], w0_ref[...], preferred_element_type=jnp.float32)
    acc = acc + jnp.dot(a1_ref[...], w1_ref[...], preferred_element_type=jnp.float32)
    o_ref[...] = jnp.tanh(acc + b_ref[...])


def _head_tm(a0, a1, w0, w1, b, *, tm_cap=512, tn_cap=512):
    M, H = a0.shape
    N = w0.shape[-1]
    tm = _tile(M, tm_cap, 8)
    tn = _tile(N, tn_cap, 128)
    grid = (1, N // tn, M // tm)
    tile_bytes = (2 * 2 * tm * H * 2 + 2 * 2 * H * tn * 2 + 2 * tn * 4
                  + 2 * tm * tn * 4)
    vmem_limit = int(min(64 * 1024 * 1024, max(16 * 1024 * 1024, 2 * tile_bytes)))
    return pl.pallas_call(
        _head_tm_body,
        out_shape=jax.ShapeDtypeStruct((1, M, N), jnp.float32),
        grid=grid,
        in_specs=[
            pl.BlockSpec((tm, H), lambda g, n, m: (m, 0)),
            pl.BlockSpec((tm, H), lambda g, n, m: (m, 0)),
            pl.BlockSpec((None, H, tn), lambda g, n, m: (g, 0, n)),
            pl.BlockSpec((None, H, tn), lambda g, n, m: (g, 0, n)),
            pl.BlockSpec((None, 1, tn), lambda g, n, m: (g, 0, n)),
        ],
        out_specs=pl.BlockSpec((None, tm, tn), lambda g, n, m: (g, m, n)),
        compiler_params=pltpu.CompilerParams(
            dimension_semantics=("parallel", "parallel", "parallel"),
            vmem_limit_bytes=vmem_limit),
    )(a0, a1, w0, w1, b)


def kernel(x, l0_fwd_wih, l0_fwd_whh, l0_fwd_b, l0_bwd_wih, l0_bwd_whh, l0_bwd_b,
           l1_fwd_wih, l1_fwd_whh, l1_fwd_b, l1_bwd_wih, l1_bwd_whh, l1_bwd_b,
           lin_w, lin_b):
    B, T, F = x.shape
    H = l0_fwd_whh.shape[0]
    N = lin_w.shape[1]
    D = N // F
    Bp = _ceil_to(B, 8)
    bf = jnp.bfloat16
    perm = functools.partial(_permute_gates, H=H)

    xt = jnp.transpose(x, (1, 0, 2))
    if Bp != B:
        xt = jnp.pad(xt, ((0, 0), (0, Bp - B), (0, 0)))

    w0 = jnp.stack([perm(l0_fwd_wih), perm(l0_bwd_wih)]).astype(bf)
    b0 = jnp.stack([perm(l0_fwd_b), perm(l0_bwd_b)])
    r0 = jnp.stack([perm(l0_fwd_whh), perm(l0_bwd_whh)]).astype(bf)
    g0 = _input_gates([xt.reshape(T * Bp, F)], [w0], b0)
    h0f, h0b = _bilstm(g0.reshape(2, T, Bp, 4 * H), r0)

    w1f, w1b = perm(l1_fwd_wih), perm(l1_bwd_wih)
    w1_lo = jnp.stack([w1f[:H], w1b[:H]]).astype(bf)
    w1_hi = jnp.stack([w1f[H:], w1b[H:]]).astype(bf)
    b1 = jnp.stack([perm(l1_fwd_b), perm(l1_bwd_b)])
    r1 = jnp.stack([perm(l1_fwd_whh), perm(l1_bwd_whh)]).astype(bf)
    g1 = _input_gates([h0f.reshape(T * Bp, H), h0b.reshape(T * Bp, H)],
                      [w1_lo, w1_hi], b1)
    h1f, h1b = _bilstm(g1.reshape(2, T, Bp, 4 * H), r1)

    Np = _ceil_to(N, 128)
    lw, lb = lin_w, lin_b
    if Np != N:
        lw = jnp.pad(lw, ((0, 0), (0, Np - N)))
        lb = jnp.pad(lb, ((0, 0), (0, Np - N)))
    y = _head_bm(h1f, h1b,
                 lw[:H][None].astype(bf), lw[H:][None].astype(bf), lb[None])
    y = y[:B, :, :N].reshape(B, T * F, D)
    return y


# speedup vs baseline: 1.0384x; 1.0384x over previous
import functools

import jax
import jax.numpy as jnp
from jax.experimental import pallas as pl
from jax.experimental.pallas import tpu as pltpu


def _ceil_to(x, m):
    return (x + m - 1) // m * m


def _tile(dim, cap, align):
    if dim <= cap:
        return dim
    t = (cap // align) * align
    while t > align and dim % t:
        t -= align
    assert dim % t == 0, (dim, cap, align)
    return t


def _div_tile(dim, cap):
    for t in range(min(dim, cap), 0, -1):
        if dim % t == 0:
            return t
    return 1


def _permute_gates(w, H):
    return jnp.concatenate([w[..., :2 * H], w[..., 3 * H:], w[..., 2 * H:3 * H]],
                           axis=-1)


def _gates_body(*refs, n_in):
    a_refs = refs[:n_in]
    w_refs = refs[n_in:2 * n_in]
    b_ref = refs[2 * n_in]
    o_ref = refs[2 * n_in + 1]
    acc = jnp.dot(a_refs[0][...].astype(jnp.bfloat16), w_refs[0][...],
                  preferred_element_type=jnp.float32)
    for i in range(1, n_in):
        acc = acc + jnp.dot(a_refs[i][...].astype(jnp.bfloat16), w_refs[i][...],
                            preferred_element_type=jnp.float32)
    o_ref[...] = (acc + b_ref[...]).astype(o_ref.dtype)


def _input_gates(a_list, w_list, b, *, tm_cap=512, tn_cap=512):
    n_in = len(a_list)
    M = a_list[0].shape[0]
    G, _, N = w_list[0].shape
    tm = _tile(M, tm_cap, 8)
    tn = _tile(N, tn_cap, 128)
    grid = (G, N // tn, M // tm)
    in_specs = []
    for a in a_list:
        in_specs.append(pl.BlockSpec((tm, a.shape[1]), lambda g, n, m: (m, 0)))
    for w in w_list:
        in_specs.append(pl.BlockSpec((None, w.shape[1], tn),
                                     lambda g, n, m: (g, 0, n)))
    in_specs.append(pl.BlockSpec((None, 1, tn), lambda g, n, m: (g, 0, n)))
    tile_bytes = (sum(2 * tm * a.shape[1] * a.dtype.itemsize for a in a_list)
                  + sum(2 * w.shape[1] * tn * 2 for w in w_list)
                  + 2 * tn * 4 + 2 * tm * tn * 2)
    vmem_limit = int(min(64 * 1024 * 1024, max(16 * 1024 * 1024, 2 * tile_bytes)))
    return pl.pallas_call(
        functools.partial(_gates_body, n_in=n_in),
        out_shape=jax.ShapeDtypeStruct((G, M, N), jnp.bfloat16),
        grid=grid,
        in_specs=in_specs,
        out_specs=pl.BlockSpec((None, tm, tn), lambda g, n, m: (g, m, n)),
        compiler_params=pltpu.CompilerParams(
            dimension_semantics=("parallel", "parallel", "parallel"),
            vmem_limit_bytes=vmem_limit),
    )(*a_list, *w_list, b)


def _lstm0_body(xf_ref, xb_ref, wih_ref, bias_ref, whh_ref, hf_ref, hb_ref,
                hf_sc, cf_sc, hb_sc, cb_sc, *, H, tc):
    @pl.when(pl.program_id(1) == 0)
    def _():
        hf_sc[...] = jnp.zeros_like(hf_sc)
        cf_sc[...] = jnp.zeros_like(cf_sc)
        hb_sc[...] = jnp.zeros_like(hb_sc)
        cb_sc[...] = jnp.zeros_like(cb_sc)

    bf16 = jnp.bfloat16
    Bh = xf_ref.shape[1]
    F = xf_ref.shape[2]
    H4 = 4 * H
    gf = (jnp.dot(xf_ref[...].reshape(tc * Bh, F).astype(bf16), wih_ref[0],
                  preferred_element_type=jnp.float32)
          + bias_ref[0]).astype(bf16).reshape(tc, Bh, H4)
    gb = (jnp.dot(xb_ref[...].reshape(tc * Bh, F).astype(bf16), wih_ref[1],
                  preferred_element_type=jnp.float32)
          + bias_ref[1]).astype(bf16).reshape(tc, Bh, H4)

    wf = whh_ref[0]
    wb = whh_ref[1]
    hf, cf = hf_sc[...], cf_sc[...]
    hb, cb = hb_sc[...], cb_sc[...]
    for t in range(tc):
        tb = tc - 1 - t
        zf = gf[t].astype(jnp.float32) + jnp.dot(
            hf.astype(bf16), wf, preferred_element_type=jnp.float32)
        zb = gb[tb].astype(jnp.float32) + jnp.dot(
            hb.astype(bf16), wb, preferred_element_type=jnp.float32)
        pf = jax.nn.sigmoid(zf[:, :3 * H])
        pb = jax.nn.sigmoid(zb[:, :3 * H])
        cf = pf[:, H:2 * H] * cf + pf[:, :H] * jnp.tanh(zf[:, 3 * H:])
        cb = pb[:, H:2 * H] * cb + pb[:, :H] * jnp.tanh(zb[:, 3 * H:])
        hf = pf[:, 2 * H:] * jnp.tanh(cf)
        hb = pb[:, 2 * H:] * jnp.tanh(cb)
        hf_ref[t] = hf.astype(bf16)
        hb_ref[tb] = hb.astype(bf16)
    hf_sc[...], cf_sc[...] = hf, cf
    hb_sc[...], cb_sc[...] = hb, cb


def _bilstm0(x_tbf, wih, bias, whh, *, tc_cap=16):
    T, Bp, F = x_tbf.shape
    H4 = whh.shape[-1]
    H = H4 // 4
    tc = _div_tile(T, tc_cap)
    nc = T // tc
    nb = 2 if Bp % 16 == 0 else 1
    Bh = Bp // nb

    out_shape = [jax.ShapeDtypeStruct((T, Bp, H), jnp.bfloat16)] * 2
    return pl.pallas_call(
        functools.partial(_lstm0_body, H=H, tc=tc),
        out_shape=out_shape,
        grid=(nb, nc),
        in_specs=[
            pl.BlockSpec((tc, Bh, F), lambda b, c: (c, b, 0)),
            pl.BlockSpec((tc, Bh, F), lambda b, c, nc=nc: (nc - 1 - c, b, 0)),
            pl.BlockSpec((2, F, H4), lambda b, c: (0, 0, 0)),
            pl.BlockSpec((2, 1, H4), lambda b, c: (0, 0, 0)),
            pl.BlockSpec((2, H, H4), lambda b, c: (0, 0, 0)),
        ],
        out_specs=[
            pl.BlockSpec((tc, Bh, H), lambda b, c: (c, b, 0)),
            pl.BlockSpec((tc, Bh, H), lambda b, c, nc=nc: (nc - 1 - c, b, 0)),
        ],
        scratch_shapes=[
            pltpu.VMEM((Bh, H), jnp.float32),
            pltpu.VMEM((Bh, H), jnp.float32),
            pltpu.VMEM((Bh, H), jnp.float32),
            pltpu.VMEM((Bh, H), jnp.float32),
        ],
        compiler_params=pltpu.CompilerParams(
            dimension_semantics=("parallel", "arbitrary")),
    )(x_tbf, x_tbf, wih, bias, whh)


def _lstm_body(gf_ref, gb_ref, whh_ref, hf_ref, hb_ref,
               hf_sc, cf_sc, hb_sc, cb_sc, *, H, tc):
    @pl.when(pl.program_id(1) == 0)
    def _():
        hf_sc[...] = jnp.zeros_like(hf_sc)
        cf_sc[...] = jnp.zeros_like(cf_sc)
        hb_sc[...] = jnp.zeros_like(hb_sc)
        cb_sc[...] = jnp.zeros_like(cb_sc)

    wf = whh_ref[0]
    wb = whh_ref[1]
    hf, cf = hf_sc[...], cf_sc[...]
    hb, cb = hb_sc[...], cb_sc[...]
    bf16 = jnp.bfloat16
    for t in range(tc):
        tb = tc - 1 - t
        zf = gf_ref[t].astype(jnp.float32) + jnp.dot(
            hf.astype(bf16), wf, preferred_element_type=jnp.float32)
        zb = gb_ref[tb].astype(jnp.float32) + jnp.dot(
            hb.astype(bf16), wb, preferred_element_type=jnp.float32)
        pf = jax.nn.sigmoid(zf[:, :3 * H])
        pb = jax.nn.sigmoid(zb[:, :3 * H])
        cf = pf[:, H:2 * H] * cf + pf[:, :H] * jnp.tanh(zf[:, 3 * H:])
        cb = pb[:, H:2 * H] * cb + pb[:, :H] * jnp.tanh(zb[:, 3 * H:])
        hf = pf[:, 2 * H:] * jnp.tanh(cf)
        hb = pb[:, 2 * H:] * jnp.tanh(cb)
        hf_ref[t] = hf.astype(bf16)
        hb_ref[tb] = hb.astype(bf16)
    hf_sc[...], cf_sc[...] = hf, cf
    hb_sc[...], cb_sc[...] = hb, cb


def _bilstm(g, whh, *, tc_cap=16):
    _, T, Bp, H4 = g.shape
    H = H4 // 4
    tc = _div_tile(T, tc_cap)
    nc = T // tc
    nb = 2 if Bp % 16 == 0 else 1
    Bh = Bp // nb

    out_shape = [jax.ShapeDtypeStruct((T, Bp, H), jnp.bfloat16)] * 2
    return pl.pallas_call(
        functools.partial(_lstm_body, H=H, tc=tc),
        out_shape=out_shape,
        grid=(nb, nc),
        in_specs=[
            pl.BlockSpec((None, tc, Bh, H4), lambda b, c: (0, c, b, 0)),
            pl.BlockSpec((None, tc, Bh, H4),
                         lambda b, c, nc=nc: (1, nc - 1 - c, b, 0)),
            pl.BlockSpec((2, H, H4), lambda b, c: (0, 0, 0)),
        ],
        out_specs=[
            pl.BlockSpec((tc, Bh, H), lambda b, c: (c, b, 0)),
            pl.BlockSpec((tc, Bh, H), lambda b, c, nc=nc: (nc - 1 - c, b, 0)),
        ],
        scratch_shapes=[
            pltpu.VMEM((Bh, H), jnp.float32),
            pltpu.VMEM((Bh, H), jnp.float32),
            pltpu.VMEM((Bh, H), jnp.float32),
            pltpu.VMEM((Bh, H), jnp.float32),
        ],
        compiler_params=pltpu.CompilerParams(
            dimension_semantics=("parallel", "arbitrary")),
    )(g, g, whh)


def _head_body(h_ref, w_ref, b_ref, o_ref, *, H, tt):
    a0 = h_ref[0].reshape(-1, H)
    a1 = h_ref[1].reshape(-1, H)
    acc = jnp.dot(a0, w_ref[0], preferred_element_type=jnp.float32)
    acc = acc + jnp.dot(a1, w_ref[1], preferred_element_type=jnp.float32)
    y = jnp.tanh(acc + b_ref[...])
    y = y.reshape(tt, -1, y.shape[-1])
    for i in range(tt):
        o_ref[:, i, :] = y[i]


def _head(h, w, b, *, tt_cap=8, tn_cap=512):
    _, T, Bp, H = h.shape
    N = w.shape[-1]
    tn = _tile(N, tn_cap, 128)
    tt = _div_tile(T, tt_cap)
    grid = (N // tn, T // tt)
    return pl.pallas_call(
        functools.partial(_head_body, H=H, tt=tt),
        out_shape=jax.ShapeDtypeStruct((Bp, T, N), jnp.float32),
        grid=grid,
        in_specs=[
            pl.BlockSpec((2, tt, Bp, H), lambda n, t: (0, t, 0, 0)),
            pl.BlockSpec((2, H, tn), lambda n, t: (0, 0, n)),
            pl.BlockSpec((1, tn), lambda n, t: (0, n)),
        ],
        out_specs=pl.BlockSpec((Bp, tt, tn), lambda n, t: (0, t, n)),
        compiler_params=pltpu.CompilerParams(
            dimension_semantics=("parallel", "parallel")),
    )(h, w, b)


def _head_bm_body(a0_ref, a1_ref, w0_ref, w1_ref, b_ref, o_ref):
    acc = jnp.dot(a0_ref[...], w0_ref[...], preferred_element_type=jnp.float32)
    acc = acc + jnp.dot(a1_ref[...], w1_ref[...], preferred_element_type=jnp.float32)
    o_ref[...] = jnp.tanh(acc + b_ref[...])


def _head_bm(hf, hb, w0, w1, b, *, tn_cap=512):
    T, Bp, H = hf.shape
    N = w0.shape[-1]
    tn = _tile(N, tn_cap, 128)
    a0 = hf.reshape(T, Bp * H)
    a1 = hb.reshape(T, Bp * H)
    grid = (N // tn, Bp)
    tile_bytes = (2 * 2 * T * H * 2 + 2 * 2 * H * tn * 2 + 2 * tn * 4
                  + 2 * T * tn * 4)
    vmem_limit = int(min(64 * 1024 * 1024, max(16 * 1024 * 1024, 2 * tile_bytes)))
    return pl.pallas_call(
        _head_bm_body,
        out_shape=jax.ShapeDtypeStruct((Bp, T, N), jnp.float32),
        grid=grid,
        in_specs=[
            pl.BlockSpec((T, H), lambda n, b: (0, b)),
            pl.BlockSpec((T, H), lambda n, b: (0, b)),
            pl.BlockSpec((None, H, tn), lambda n, b: (0, 0, n)),
            pl.BlockSpec((None, H, tn), lambda n, b: (0, 0, n)),
            pl.BlockSpec((None, 1, tn), lambda n, b: (0, 0, n)),
        ],
        out_specs=pl.BlockSpec((None, T, tn), lambda n, b: (b, 0, n)),
        compiler_params=pltpu.CompilerParams(
            dimension_semantics=("parallel", "parallel"),
            vmem_limit_bytes=vmem_limit),
    )(a0, a1, w0, w1, b)


def _head_tm_body(a0_ref, a1_ref, w0_ref, w1_ref, b_ref, o_ref):
    acc = jnp.dot(a0_ref[...], w0_ref[...], preferred_element_type=jnp.float32)
    acc = acc + jnp.dot(a1_ref[...], w1_ref[...], preferred_element_type=jnp.float32)
    o_ref[...] = jnp.tanh(acc + b_ref[...])


def _head_tm(a0, a1, w0, w1, b, *, tm_cap=512, tn_cap=512):
    M, H = a0.shape
    N = w0.shape[-1]
    tm = _tile(M, tm_cap, 8)
    tn = _tile(N, tn_cap, 128)
    grid = (1, N // tn, M // tm)
    tile_bytes = (2 * 2 * tm * H * 2 + 2 * 2 * H * tn * 2 + 2 * tn * 4
                  + 2 * tm * tn * 4)
    vmem_limit = int(min(64 * 1024 * 1024, max(16 * 1024 * 1024, 2 * tile_bytes)))
    return pl.pallas_call(
        _head_tm_body,
        out_shape=jax.ShapeDtypeStruct((1, M, N), jnp.float32),
        grid=grid,
        in_specs=[
            pl.BlockSpec((tm, H), lambda g, n, m: (m, 0)),
            pl.BlockSpec((tm, H), lambda g, n, m: (m, 0)),
            pl.BlockSpec((None, H, tn), lambda g, n, m: (g, 0, n)),
            pl.BlockSpec((None, H, tn), lambda g, n, m: (g, 0, n)),
            pl.BlockSpec((None, 1, tn), lambda g, n, m: (g, 0, n)),
        ],
        out_specs=pl.BlockSpec((None, tm, tn), lambda g, n, m: (g, m, n)),
        compiler_params=pltpu.CompilerParams(
            dimension_semantics=("parallel", "parallel", "parallel"),
            vmem_limit_bytes=vmem_limit),
    )(a0, a1, w0, w1, b)


def kernel(x, l0_fwd_wih, l0_fwd_whh, l0_fwd_b, l0_bwd_wih, l0_bwd_whh, l0_bwd_b,
           l1_fwd_wih, l1_fwd_whh, l1_fwd_b, l1_bwd_wih, l1_bwd_whh, l1_bwd_b,
           lin_w, lin_b):
    B, T, F = x.shape
    H = l0_fwd_whh.shape[0]
    N = lin_w.shape[1]
    D = N // F
    Bp = _ceil_to(B, 8)
    bf = jnp.bfloat16
    perm = functools.partial(_permute_gates, H=H)

    xt = jnp.transpose(x, (1, 0, 2))
    if Bp != B:
        xt = jnp.pad(xt, ((0, 0), (0, Bp - B), (0, 0)))

    w0 = jnp.stack([perm(l0_fwd_wih), perm(l0_bwd_wih)]).astype(bf)
    b0 = jnp.stack([perm(l0_fwd_b), perm(l0_bwd_b)])
    r0 = jnp.stack([perm(l0_fwd_whh), perm(l0_bwd_whh)]).astype(bf)
    h0f, h0b = _bilstm0(xt, w0, b0, r0)

    w1f, w1b = perm(l1_fwd_wih), perm(l1_bwd_wih)
    w1_lo = jnp.stack([w1f[:H], w1b[:H]]).astype(bf)
    w1_hi = jnp.stack([w1f[H:], w1b[H:]]).astype(bf)
    b1 = jnp.stack([perm(l1_fwd_b), perm(l1_bwd_b)])
    r1 = jnp.stack([perm(l1_fwd_whh), perm(l1_bwd_whh)]).astype(bf)
    g1 = _input_gates([h0f.reshape(T * Bp, H), h0b.reshape(T * Bp, H)],
                      [w1_lo, w1_hi], b1)
    h1f, h1b = _bilstm(g1.reshape(2, T, Bp, 4 * H), r1)

    Np = _ceil_to(N, 128)
    lw, lb = lin_w, lin_b
    if Np != N:
        lw = jnp.pad(lw, ((0, 0), (0, Np - N)))
        lb = jnp.pad(lb, ((0, 0), (0, Np - N)))
    y = _head_bm(h1f, h1b,
                 lw[:H][None].astype(bf), lw[H:][None].astype(bf), lb[None])
    y = y[:B, :, :N].reshape(B, T * F, D)
    return y


# speedup vs baseline: 1.3767x; 1.3258x over previous
import functools

import jax
import jax.numpy as jnp
from jax.experimental import pallas as pl
from jax.experimental.pallas import tpu as pltpu


def _ceil_to(x, m):
    return (x + m - 1) // m * m


def _tile(dim, cap, align):
    if dim <= cap:
        return dim
    t = (cap // align) * align
    while t > align and dim % t:
        t -= align
    assert dim % t == 0, (dim, cap, align)
    return t


def _div_tile(dim, cap):
    for t in range(min(dim, cap), 0, -1):
        if dim % t == 0:
            return t
    return 1


def _permute_gates(w, H):
    return jnp.concatenate([w[..., :2 * H], w[..., 3 * H:], w[..., 2 * H:3 * H]],
                           axis=-1)


def _gates_body(*refs, n_in):
    a_refs = refs[:n_in]
    w_refs = refs[n_in:2 * n_in]
    b_ref = refs[2 * n_in]
    o_ref = refs[2 * n_in + 1]
    acc = jnp.dot(a_refs[0][...].astype(jnp.bfloat16), w_refs[0][...],
                  preferred_element_type=jnp.float32)
    for i in range(1, n_in):
        acc = acc + jnp.dot(a_refs[i][...].astype(jnp.bfloat16), w_refs[i][...],
                            preferred_element_type=jnp.float32)
    o_ref[...] = (acc + b_ref[...]).astype(o_ref.dtype)


def _input_gates(a_list, w_list, b, *, tm_cap=512, tn_cap=512):
    n_in = len(a_list)
    M = a_list[0].shape[0]
    G, _, N = w_list[0].shape
    tm = _tile(M, tm_cap, 8)
    tn = _tile(N, tn_cap, 128)
    grid = (G, N // tn, M // tm)
    in_specs = []
    for a in a_list:
        in_specs.append(pl.BlockSpec((tm, a.shape[1]), lambda g, n, m: (m, 0)))
    for w in w_list:
        in_specs.append(pl.BlockSpec((None, w.shape[1], tn),
                                     lambda g, n, m: (g, 0, n)))
    in_specs.append(pl.BlockSpec((None, 1, tn), lambda g, n, m: (g, 0, n)))
    tile_bytes = (sum(2 * tm * a.shape[1] * a.dtype.itemsize for a in a_list)
                  + sum(2 * w.shape[1] * tn * 2 for w in w_list)
                  + 2 * tn * 4 + 2 * tm * tn * 2)
    vmem_limit = int(min(64 * 1024 * 1024, max(16 * 1024 * 1024, 2 * tile_bytes)))
    return pl.pallas_call(
        functools.partial(_gates_body, n_in=n_in),
        out_shape=jax.ShapeDtypeStruct((G, M, N), jnp.bfloat16),
        grid=grid,
        in_specs=in_specs,
        out_specs=pl.BlockSpec((None, tm, tn), lambda g, n, m: (g, m, n)),
        compiler_params=pltpu.CompilerParams(
            dimension_semantics=("parallel", "parallel", "parallel"),
            vmem_limit_bytes=vmem_limit),
    )(*a_list, *w_list, b)


def _lstm0_body(xf_ref, xb_ref, wih_ref, bias_ref, whh_ref, hf_ref, hb_ref,
                hf_sc, cf_sc, hb_sc, cb_sc, *, H, tc):
    @pl.when(pl.program_id(1) == 0)
    def _():
        hf_sc[...] = jnp.zeros_like(hf_sc)
        cf_sc[...] = jnp.zeros_like(cf_sc)
        hb_sc[...] = jnp.zeros_like(hb_sc)
        cb_sc[...] = jnp.zeros_like(cb_sc)

    bf16 = jnp.bfloat16
    Bh = xf_ref.shape[1]
    F = xf_ref.shape[2]
    H4 = 4 * H
    gf = (jnp.dot(xf_ref[...].reshape(tc * Bh, F).astype(bf16), wih_ref[0],
                  preferred_element_type=jnp.float32)
          + bias_ref[0]).astype(bf16).reshape(tc, Bh, H4)
    gb = (jnp.dot(xb_ref[...].reshape(tc * Bh, F).astype(bf16), wih_ref[1],
                  preferred_element_type=jnp.float32)
          + bias_ref[1]).astype(bf16).reshape(tc, Bh, H4)

    wf = whh_ref[0]
    wb = whh_ref[1]
    hf, cf = hf_sc[...], cf_sc[...]
    hb, cb = hb_sc[...], cb_sc[...]
    for t in range(tc):
        tb = tc - 1 - t
        zf = gf[t].astype(jnp.float32) + jnp.dot(
            hf.astype(bf16), wf, preferred_element_type=jnp.float32)
        zb = gb[tb].astype(jnp.float32) + jnp.dot(
            hb.astype(bf16), wb, preferred_element_type=jnp.float32)
        pf = jax.nn.sigmoid(zf[:, :3 * H])
        pb = jax.nn.sigmoid(zb[:, :3 * H])
        cf = pf[:, H:2 * H] * cf + pf[:, :H] * jnp.tanh(zf[:, 3 * H:])
        cb = pb[:, H:2 * H] * cb + pb[:, :H] * jnp.tanh(zb[:, 3 * H:])
        hf = pf[:, 2 * H:] * jnp.tanh(cf)
        hb = pb[:, 2 * H:] * jnp.tanh(cb)
        hf_ref[t] = hf.astype(bf16)
        hb_ref[tb] = hb.astype(bf16)
    hf_sc[...], cf_sc[...] = hf, cf
    hb_sc[...], cb_sc[...] = hb, cb


def _bilstm0(x_tbf, wih, bias, whh, *, tc_cap=16):
    T, Bp, F = x_tbf.shape
    H4 = whh.shape[-1]
    H = H4 // 4
    tc = _div_tile(T, tc_cap)
    nc = T // tc
    nb = 2 if Bp % 16 == 0 else 1
    Bh = Bp // nb

    out_shape = [jax.ShapeDtypeStruct((T, Bp, H), jnp.bfloat16)] * 2
    return pl.pallas_call(
        functools.partial(_lstm0_body, H=H, tc=tc),
        out_shape=out_shape,
        grid=(nb, nc),
        in_specs=[
            pl.BlockSpec((tc, Bh, F), lambda b, c: (c, b, 0)),
            pl.BlockSpec((tc, Bh, F), lambda b, c, nc=nc: (nc - 1 - c, b, 0)),
            pl.BlockSpec((2, F, H4), lambda b, c: (0, 0, 0)),
            pl.BlockSpec((2, 1, H4), lambda b, c: (0, 0, 0)),
            pl.BlockSpec((2, H, H4), lambda b, c: (0, 0, 0)),
        ],
        out_specs=[
            pl.BlockSpec((tc, Bh, H), lambda b, c: (c, b, 0)),
            pl.BlockSpec((tc, Bh, H), lambda b, c, nc=nc: (nc - 1 - c, b, 0)),
        ],
        scratch_shapes=[
            pltpu.VMEM((Bh, H), jnp.float32),
            pltpu.VMEM((Bh, H), jnp.float32),
            pltpu.VMEM((Bh, H), jnp.float32),
            pltpu.VMEM((Bh, H), jnp.float32),
        ],
        compiler_params=pltpu.CompilerParams(
            dimension_semantics=("parallel", "arbitrary")),
    )(x_tbf, x_tbf, wih, bias, whh)


def _lstm_body(gf_ref, gb_ref, whh_ref, hf_ref, hb_ref,
               hf_sc, cf_sc, hb_sc, cb_sc, *, H, tc):
    @pl.when(pl.program_id(1) == 0)
    def _():
        hf_sc[...] = jnp.zeros_like(hf_sc)
        cf_sc[...] = jnp.zeros_like(cf_sc)
        hb_sc[...] = jnp.zeros_like(hb_sc)
        cb_sc[...] = jnp.zeros_like(cb_sc)

    wf = whh_ref[0]
    wb = whh_ref[1]
    hf, cf = hf_sc[...], cf_sc[...]
    hb, cb = hb_sc[...], cb_sc[...]
    bf16 = jnp.bfloat16
    for t in range(tc):
        tb = tc - 1 - t
        zf = gf_ref[t].astype(jnp.float32) + jnp.dot(
            hf.astype(bf16), wf, preferred_element_type=jnp.float32)
        zb = gb_ref[tb].astype(jnp.float32) + jnp.dot(
            hb.astype(bf16), wb, preferred_element_type=jnp.float32)
        pf = jax.nn.sigmoid(zf[:, :3 * H])
        pb = jax.nn.sigmoid(zb[:, :3 * H])
        cf = pf[:, H:2 * H] * cf + pf[:, :H] * jnp.tanh(zf[:, 3 * H:])
        cb = pb[:, H:2 * H] * cb + pb[:, :H] * jnp.tanh(zb[:, 3 * H:])
        hf = pf[:, 2 * H:] * jnp.tanh(cf)
        hb = pb[:, 2 * H:] * jnp.tanh(cb)
        hf_ref[t] = hf.astype(bf16)
        hb_ref[tb] = hb.astype(bf16)
    hf_sc[...], cf_sc[...] = hf, cf
    hb_sc[...], cb_sc[...] = hb, cb


def _bilstm(g, whh, *, tc_cap=16):
    _, T, Bp, H4 = g.shape
    H = H4 // 4
    tc = _div_tile(T, tc_cap)
    nc = T // tc
    nb = 2 if Bp % 16 == 0 else 1
    Bh = Bp // nb

    out_shape = [jax.ShapeDtypeStruct((T, Bp, H), jnp.bfloat16)] * 2
    return pl.pallas_call(
        functools.partial(_lstm_body, H=H, tc=tc),
        out_shape=out_shape,
        grid=(nb, nc),
        in_specs=[
            pl.BlockSpec((None, tc, Bh, H4), lambda b, c: (0, c, b, 0)),
            pl.BlockSpec((None, tc, Bh, H4),
                         lambda b, c, nc=nc: (1, nc - 1 - c, b, 0)),
            pl.BlockSpec((2, H, H4), lambda b, c: (0, 0, 0)),
        ],
        out_specs=[
            pl.BlockSpec((tc, Bh, H), lambda b, c: (c, b, 0)),
            pl.BlockSpec((tc, Bh, H), lambda b, c, nc=nc: (nc - 1 - c, b, 0)),
        ],
        scratch_shapes=[
            pltpu.VMEM((Bh, H), jnp.float32),
            pltpu.VMEM((Bh, H), jnp.float32),
            pltpu.VMEM((Bh, H), jnp.float32),
            pltpu.VMEM((Bh, H), jnp.float32),
        ],
        compiler_params=pltpu.CompilerParams(
            dimension_semantics=("parallel", "arbitrary")),
    )(g, g, whh)


def _head_body(h_ref, w_ref, b_ref, o_ref, *, H, tt):
    a0 = h_ref[0].reshape(-1, H)
    a1 = h_ref[1].reshape(-1, H)
    acc = jnp.dot(a0, w_ref[0], preferred_element_type=jnp.float32)
    acc = acc + jnp.dot(a1, w_ref[1], preferred_element_type=jnp.float32)
    y = jnp.tanh(acc + b_ref[...])
    y = y.reshape(tt, -1, y.shape[-1])
    for i in range(tt):
        o_ref[:, i, :] = y[i]


def _head(h, w, b, *, tt_cap=8, tn_cap=512):
    _, T, Bp, H = h.shape
    N = w.shape[-1]
    tn = _tile(N, tn_cap, 128)
    tt = _div_tile(T, tt_cap)
    grid = (N // tn, T // tt)
    return pl.pallas_call(
        functools.partial(_head_body, H=H, tt=tt),
        out_shape=jax.ShapeDtypeStruct((Bp, T, N), jnp.float32),
        grid=grid,
        in_specs=[
            pl.BlockSpec((2, tt, Bp, H), lambda n, t: (0, t, 0, 0)),
            pl.BlockSpec((2, H, tn), lambda n, t: (0, 0, n)),
            pl.BlockSpec((1, tn), lambda n, t: (0, n)),
        ],
        out_specs=pl.BlockSpec((Bp, tt, tn), lambda n, t: (0, t, n)),
        compiler_params=pltpu.CompilerParams(
            dimension_semantics=("parallel", "parallel")),
    )(h, w, b)


def _head_bm_body(a0_ref, a1_ref, w0_ref, w1_ref, b_ref, o_ref):
    acc = jnp.dot(a0_ref[...], w0_ref[...], preferred_element_type=jnp.float32)
    acc = acc + jnp.dot(a1_ref[...], w1_ref[...], preferred_element_type=jnp.float32)
    o_ref[...] = jnp.tanh(acc + b_ref[...])


def _head_bm(hf, hb, w0, w1, b, *, tn_cap=512):
    T, Bp, H = hf.shape
    N = w0.shape[-1]
    tn = _tile(N, tn_cap, 128)
    a0 = hf.reshape(T, Bp * H)
    a1 = hb.reshape(T, Bp * H)
    grid = (N // tn, Bp)
    tile_bytes = (2 * 2 * T * H * 2 + 2 * 2 * H * tn * 2 + 2 * tn * 4
                  + 2 * T * tn * 4)
    vmem_limit = int(min(64 * 1024 * 1024, max(16 * 1024 * 1024, 2 * tile_bytes)))
    return pl.pallas_call(
        _head_bm_body,
        out_shape=jax.ShapeDtypeStruct((Bp, T, N), jnp.float32),
        grid=grid,
        in_specs=[
            pl.BlockSpec((T, H), lambda n, b: (0, b)),
            pl.BlockSpec((T, H), lambda n, b: (0, b)),
            pl.BlockSpec((None, H, tn), lambda n, b: (0, 0, n)),
            pl.BlockSpec((None, H, tn), lambda n, b: (0, 0, n)),
            pl.BlockSpec((None, 1, tn), lambda n, b: (0, 0, n)),
        ],
        out_specs=pl.BlockSpec((None, T, tn), lambda n, b: (b, 0, n)),
        compiler_params=pltpu.CompilerParams(
            dimension_semantics=("parallel", "parallel"),
            vmem_limit_bytes=vmem_limit),
    )(a0, a1, w0, w1, b)


def _head_tm_body(a0_ref, a1_ref, w0_ref, w1_ref, b_ref, o_ref):
    acc = jnp.dot(a0_ref[...], w0_ref[...], preferred_element_type=jnp.float32)
    acc = acc + jnp.dot(a1_ref[...], w1_ref[...], preferred_element_type=jnp.float32)
    o_ref[...] = jnp.tanh(acc + b_ref[...])


def _head_tm(a0, a1, w0, w1, b, *, tm_cap=512, tn_cap=512):
    M, H = a0.shape
    N = w0.shape[-1]
    tm = _tile(M, tm_cap, 8)
    tn = _tile(N, tn_cap, 128)
    grid = (1, N // tn, M // tm)
    tile_bytes = (2 * 2 * tm * H * 2 + 2 * 2 * H * tn * 2 + 2 * tn * 4
                  + 2 * tm * tn * 4)
    vmem_limit = int(min(64 * 1024 * 1024, max(16 * 1024 * 1024, 2 * tile_bytes)))
    return pl.pallas_call(
        _head_tm_body,
        out_shape=jax.ShapeDtypeStruct((1, M, N), jnp.float32),
        grid=grid,
        in_specs=[
            pl.BlockSpec((tm, H), lambda g, n, m: (m, 0)),
            pl.BlockSpec((tm, H), lambda g, n, m: (m, 0)),
            pl.BlockSpec((None, H, tn), lambda g, n, m: (g, 0, n)),
            pl.BlockSpec((None, H, tn), lambda g, n, m: (g, 0, n)),
            pl.BlockSpec((None, 1, tn), lambda g, n, m: (g, 0, n)),
        ],
        out_specs=pl.BlockSpec((None, tm, tn), lambda g, n, m: (g, m, n)),
        compiler_params=pltpu.CompilerParams(
            dimension_semantics=("parallel", "parallel", "parallel"),
            vmem_limit_bytes=vmem_limit),
    )(a0, a1, w0, w1, b)


def kernel(x, l0_fwd_wih, l0_fwd_whh, l0_fwd_b, l0_bwd_wih, l0_bwd_whh, l0_bwd_b,
           l1_fwd_wih, l1_fwd_whh, l1_fwd_b, l1_bwd_wih, l1_bwd_whh, l1_bwd_b,
           lin_w, lin_b):
    B, T, F = x.shape
    H = l0_fwd_whh.shape[0]
    N = lin_w.shape[1]
    D = N // F
    Bp = _ceil_to(B, 8)
    bf = jnp.bfloat16
    perm = functools.partial(_permute_gates, H=H)

    xt = jnp.transpose(x, (1, 0, 2))
    if Bp != B:
        xt = jnp.pad(xt, ((0, 0), (0, Bp - B), (0, 0)))

    w0 = jnp.stack([perm(l0_fwd_wih), perm(l0_bwd_wih)]).astype(bf)
    b0 = jnp.stack([perm(l0_fwd_b), perm(l0_bwd_b)])
    r0 = jnp.stack([perm(l0_fwd_whh), perm(l0_bwd_whh)]).astype(bf)
    h0f, h0b = _bilstm0(xt, w0, b0, r0)

    w1f, w1b = perm(l1_fwd_wih), perm(l1_bwd_wih)
    w1_lo = jnp.stack([w1f[:H], w1b[:H]]).astype(bf)
    w1_hi = jnp.stack([w1f[H:], w1b[H:]]).astype(bf)
    b1 = jnp.stack([perm(l1_fwd_b), perm(l1_bwd_b)])
    r1 = jnp.stack([perm(l1_fwd_whh), perm(l1_bwd_whh)]).astype(bf)
    g1 = _input_gates([h0f.reshape(T * Bp, H), h0b.reshape(T * Bp, H)],
                      [w1_lo, w1_hi], b1)
    h1f, h1b = _bilstm(g1.reshape(2, T, Bp, 4 * H), r1)

    Np = _ceil_to(N, 128)
    lw, lb = lin_w, lin_b
    if Np != N:
        lw = jnp.pad(lw, ((0, 0), (0, Np - N)))
        lb = jnp.pad(lb, ((0, 0), (0, Np - N)))
    y = _head_tm(h1f.reshape(T * Bp, H), h1b.reshape(T * Bp, H),
                 lw[:H][None].astype(bf), lw[H:][None].astype(bf), lb[None])
    y = y[0][:, :N].reshape(T, Bp, N)
    y = jnp.transpose(y, (1, 0, 2))[:B].reshape(B, T * F, D)
    return y


# speedup vs baseline: 1.4510x; 1.0540x over previous
import functools

import jax
import jax.numpy as jnp
from jax.experimental import pallas as pl
from jax.experimental.pallas import tpu as pltpu


def _ceil_to(x, m):
    return (x + m - 1) // m * m


def _tile(dim, cap, align):
    if dim <= cap:
        return dim
    t = (cap // align) * align
    while t > align and dim % t:
        t -= align
    assert dim % t == 0, (dim, cap, align)
    return t


def _div_tile(dim, cap):
    for t in range(min(dim, cap), 0, -1):
        if dim % t == 0:
            return t
    return 1


def _permute_gates(w, H):
    return jnp.concatenate([w[..., :2 * H], w[..., 3 * H:], w[..., 2 * H:3 * H]],
                           axis=-1)


def _gates_body(*refs, n_in):
    a_refs = refs[:n_in]
    w_refs = refs[n_in:2 * n_in]
    b_ref = refs[2 * n_in]
    o_ref = refs[2 * n_in + 1]
    acc = jnp.dot(a_refs[0][...].astype(jnp.bfloat16), w_refs[0][...],
                  preferred_element_type=jnp.float32)
    for i in range(1, n_in):
        acc = acc + jnp.dot(a_refs[i][...].astype(jnp.bfloat16), w_refs[i][...],
                            preferred_element_type=jnp.float32)
    o_ref[...] = (acc + b_ref[...]).astype(o_ref.dtype)


def _input_gates(a_list, w_list, b, *, tm_cap=512, tn_cap=512):
    n_in = len(a_list)
    M = a_list[0].shape[0]
    G, _, N = w_list[0].shape
    tm = _tile(M, tm_cap, 8)
    tn = _tile(N, tn_cap, 128)
    grid = (G, N // tn, M // tm)
    in_specs = []
    for a in a_list:
        in_specs.append(pl.BlockSpec((tm, a.shape[1]), lambda g, n, m: (m, 0)))
    for w in w_list:
        in_specs.append(pl.BlockSpec((None, w.shape[1], tn),
                                     lambda g, n, m: (g, 0, n)))
    in_specs.append(pl.BlockSpec((None, 1, tn), lambda g, n, m: (g, 0, n)))
    tile_bytes = (sum(2 * tm * a.shape[1] * a.dtype.itemsize for a in a_list)
                  + sum(2 * w.shape[1] * tn * 2 for w in w_list)
                  + 2 * tn * 4 + 2 * tm * tn * 2)
    vmem_limit = int(min(64 * 1024 * 1024, max(16 * 1024 * 1024, 2 * tile_bytes)))
    return pl.pallas_call(
        functools.partial(_gates_body, n_in=n_in),
        out_shape=jax.ShapeDtypeStruct((G, M, N), jnp.bfloat16),
        grid=grid,
        in_specs=in_specs,
        out_specs=pl.BlockSpec((None, tm, tn), lambda g, n, m: (g, m, n)),
        compiler_params=pltpu.CompilerParams(
            dimension_semantics=("parallel", "parallel", "parallel"),
            vmem_limit_bytes=vmem_limit),
    )(*a_list, *w_list, b)


def _lstm0_body(xf_ref, xb_ref, wih_ref, bias_ref, whh_ref, hf_ref, hb_ref,
                hf_sc, cf_sc, hb_sc, cb_sc, *, H, tc):
    @pl.when(pl.program_id(1) == 0)
    def _():
        hf_sc[...] = jnp.zeros_like(hf_sc)
        cf_sc[...] = jnp.zeros_like(cf_sc)
        hb_sc[...] = jnp.zeros_like(hb_sc)
        cb_sc[...] = jnp.zeros_like(cb_sc)

    bf16 = jnp.bfloat16
    Bh = xf_ref.shape[1]
    F = xf_ref.shape[2]
    H4 = 4 * H
    gf = (jnp.dot(xf_ref[...].reshape(tc * Bh, F).astype(bf16), wih_ref[0],
                  preferred_element_type=jnp.float32)
          + bias_ref[0]).astype(bf16).reshape(tc, Bh, H4)
    gb = (jnp.dot(xb_ref[...].reshape(tc * Bh, F).astype(bf16), wih_ref[1],
                  preferred_element_type=jnp.float32)
          + bias_ref[1]).astype(bf16).reshape(tc, Bh, H4)

    wf = whh_ref[0]
    wb = whh_ref[1]
    hf, cf = hf_sc[...], cf_sc[...]
    hb, cb = hb_sc[...], cb_sc[...]
    for t in range(tc):
        tb = tc - 1 - t
        zf = gf[t].astype(jnp.float32) + jnp.dot(
            hf.astype(bf16), wf, preferred_element_type=jnp.float32)
        zb = gb[tb].astype(jnp.float32) + jnp.dot(
            hb.astype(bf16), wb, preferred_element_type=jnp.float32)
        pf = jax.nn.sigmoid(zf[:, :3 * H])
        pb = jax.nn.sigmoid(zb[:, :3 * H])
        cf = pf[:, H:2 * H] * cf + pf[:, :H] * jnp.tanh(zf[:, 3 * H:])
        cb = pb[:, H:2 * H] * cb + pb[:, :H] * jnp.tanh(zb[:, 3 * H:])
        hf = pf[:, 2 * H:] * jnp.tanh(cf)
        hb = pb[:, 2 * H:] * jnp.tanh(cb)
        hf_ref[t] = hf.astype(bf16)
        hb_ref[tb] = hb.astype(bf16)
    hf_sc[...], cf_sc[...] = hf, cf
    hb_sc[...], cb_sc[...] = hb, cb


def _bilstm0(x_tbf, wih, bias, whh, *, tc_cap=16):
    T, Bp, F = x_tbf.shape
    H4 = whh.shape[-1]
    H = H4 // 4
    tc = _div_tile(T, tc_cap)
    nc = T // tc
    nb = 2 if Bp % 16 == 0 else 1
    Bh = Bp // nb

    out_shape = [jax.ShapeDtypeStruct((T, Bp, H), jnp.bfloat16)] * 2
    return pl.pallas_call(
        functools.partial(_lstm0_body, H=H, tc=tc),
        out_shape=out_shape,
        grid=(nb, nc),
        in_specs=[
            pl.BlockSpec((tc, Bh, F), lambda b, c: (c, b, 0)),
            pl.BlockSpec((tc, Bh, F), lambda b, c, nc=nc: (nc - 1 - c, b, 0)),
            pl.BlockSpec((2, F, H4), lambda b, c: (0, 0, 0)),
            pl.BlockSpec((2, 1, H4), lambda b, c: (0, 0, 0)),
            pl.BlockSpec((2, H, H4), lambda b, c: (0, 0, 0)),
        ],
        out_specs=[
            pl.BlockSpec((tc, Bh, H), lambda b, c: (c, b, 0)),
            pl.BlockSpec((tc, Bh, H), lambda b, c, nc=nc: (nc - 1 - c, b, 0)),
        ],
        scratch_shapes=[
            pltpu.VMEM((Bh, H), jnp.float32),
            pltpu.VMEM((Bh, H), jnp.float32),
            pltpu.VMEM((Bh, H), jnp.float32),
            pltpu.VMEM((Bh, H), jnp.float32),
        ],
        compiler_params=pltpu.CompilerParams(
            dimension_semantics=("parallel", "arbitrary")),
    )(x_tbf, x_tbf, wih, bias, whh)


def _lstm1_body(af_ref, bf_ref, ab_ref, bb_ref, wih_ref, bias_ref, whh_ref,
                hf_ref, hb_ref, hf_sc, cf_sc, hb_sc, cb_sc, *, H, tc):
    @pl.when(pl.program_id(1) == 0)
    def _():
        hf_sc[...] = jnp.zeros_like(hf_sc)
        cf_sc[...] = jnp.zeros_like(cf_sc)
        hb_sc[...] = jnp.zeros_like(hb_sc)
        cb_sc[...] = jnp.zeros_like(cb_sc)

    bf16 = jnp.bfloat16
    Bh = af_ref.shape[1]
    H4 = 4 * H
    gf = (jnp.dot(af_ref[...].reshape(tc * Bh, H), wih_ref[0, :H],
                  preferred_element_type=jnp.float32)
          + jnp.dot(bf_ref[...].reshape(tc * Bh, H), wih_ref[0, H:],
                    preferred_element_type=jnp.float32)
          + bias_ref[0]).astype(bf16).reshape(tc, Bh, H4)
    gb = (jnp.dot(ab_ref[...].reshape(tc * Bh, H), wih_ref[1, :H],
                  preferred_element_type=jnp.float32)
          + jnp.dot(bb_ref[...].reshape(tc * Bh, H), wih_ref[1, H:],
                    preferred_element_type=jnp.float32)
          + bias_ref[1]).astype(bf16).reshape(tc, Bh, H4)

    wf = whh_ref[0]
    wb = whh_ref[1]
    hf, cf = hf_sc[...], cf_sc[...]
    hb, cb = hb_sc[...], cb_sc[...]
    for t in range(tc):
        tb = tc - 1 - t
        zf = gf[t].astype(jnp.float32) + jnp.dot(
            hf.astype(bf16), wf, preferred_element_type=jnp.float32)
        zb = gb[tb].astype(jnp.float32) + jnp.dot(
            hb.astype(bf16), wb, preferred_element_type=jnp.float32)
        pf = jax.nn.sigmoid(zf[:, :3 * H])
        pb = jax.nn.sigmoid(zb[:, :3 * H])
        cf = pf[:, H:2 * H] * cf + pf[:, :H] * jnp.tanh(zf[:, 3 * H:])
        cb = pb[:, H:2 * H] * cb + pb[:, :H] * jnp.tanh(zb[:, 3 * H:])
        hf = pf[:, 2 * H:] * jnp.tanh(cf)
        hb = pb[:, 2 * H:] * jnp.tanh(cb)
        hf_ref[t] = hf.astype(bf16)
        hb_ref[tb] = hb.astype(bf16)
    hf_sc[...], cf_sc[...] = hf, cf
    hb_sc[...], cb_sc[...] = hb, cb


def _bilstm1(h0f, h0b, wih, bias, whh, *, tc_cap=16):
    T, Bp, H = h0f.shape
    H4 = whh.shape[-1]
    tc = _div_tile(T, tc_cap)
    nc = T // tc
    nb = 2 if Bp % 16 == 0 else 1
    Bh = Bp // nb

    fwd = lambda b, c: (c, b, 0)
    bwd = lambda b, c, nc=nc: (nc - 1 - c, b, 0)
    out_shape = [jax.ShapeDtypeStruct((T, Bp, H), jnp.bfloat16)] * 2
    return pl.pallas_call(
        functools.partial(_lstm1_body, H=H, tc=tc),
        out_shape=out_shape,
        grid=(nb, nc),
        in_specs=[
            pl.BlockSpec((tc, Bh, H), fwd),
            pl.BlockSpec((tc, Bh, H), fwd),
            pl.BlockSpec((tc, Bh, H), bwd),
            pl.BlockSpec((tc, Bh, H), bwd),
            pl.BlockSpec((2, 2 * H, H4), lambda b, c: (0, 0, 0)),
            pl.BlockSpec((2, 1, H4), lambda b, c: (0, 0, 0)),
            pl.BlockSpec((2, H, H4), lambda b, c: (0, 0, 0)),
        ],
        out_specs=[
            pl.BlockSpec((tc, Bh, H), fwd),
            pl.BlockSpec((tc, Bh, H), bwd),
        ],
        scratch_shapes=[
            pltpu.VMEM((Bh, H), jnp.float32),
            pltpu.VMEM((Bh, H), jnp.float32),
            pltpu.VMEM((Bh, H), jnp.float32),
            pltpu.VMEM((Bh, H), jnp.float32),
        ],
        compiler_params=pltpu.CompilerParams(
            dimension_semantics=("parallel", "arbitrary")),
    )(h0f, h0b, h0f, h0b, wih, bias, whh)


def _lstm_body(gf_ref, gb_ref, whh_ref, hf_ref, hb_ref,
               hf_sc, cf_sc, hb_sc, cb_sc, *, H, tc):
    @pl.when(pl.program_id(1) == 0)
    def _():
        hf_sc[...] = jnp.zeros_like(hf_sc)
        cf_sc[...] = jnp.zeros_like(cf_sc)
        hb_sc[...] = jnp.zeros_like(hb_sc)
        cb_sc[...] = jnp.zeros_like(cb_sc)

    wf = whh_ref[0]
    wb = whh_ref[1]
    hf, cf = hf_sc[...], cf_sc[...]
    hb, cb = hb_sc[...], cb_sc[...]
    bf16 = jnp.bfloat16
    for t in range(tc):
        tb = tc - 1 - t
        zf = gf_ref[t].astype(jnp.float32) + jnp.dot(
            hf.astype(bf16), wf, preferred_element_type=jnp.float32)
        zb = gb_ref[tb].astype(jnp.float32) + jnp.dot(
            hb.astype(bf16), wb, preferred_element_type=jnp.float32)
        pf = jax.nn.sigmoid(zf[:, :3 * H])
        pb = jax.nn.sigmoid(zb[:, :3 * H])
        cf = pf[:, H:2 * H] * cf + pf[:, :H] * jnp.tanh(zf[:, 3 * H:])
        cb = pb[:, H:2 * H] * cb + pb[:, :H] * jnp.tanh(zb[:, 3 * H:])
        hf = pf[:, 2 * H:] * jnp.tanh(cf)
        hb = pb[:, 2 * H:] * jnp.tanh(cb)
        hf_ref[t] = hf.astype(bf16)
        hb_ref[tb] = hb.astype(bf16)
    hf_sc[...], cf_sc[...] = hf, cf
    hb_sc[...], cb_sc[...] = hb, cb


def _bilstm(g, whh, *, tc_cap=16):
    _, T, Bp, H4 = g.shape
    H = H4 // 4
    tc = _div_tile(T, tc_cap)
    nc = T // tc
    nb = 2 if Bp % 16 == 0 else 1
    Bh = Bp // nb

    out_shape = [jax.ShapeDtypeStruct((T, Bp, H), jnp.bfloat16)] * 2
    return pl.pallas_call(
        functools.partial(_lstm_body, H=H, tc=tc),
        out_shape=out_shape,
        grid=(nb, nc),
        in_specs=[
            pl.BlockSpec((None, tc, Bh, H4), lambda b, c: (0, c, b, 0)),
            pl.BlockSpec((None, tc, Bh, H4),
                         lambda b, c, nc=nc: (1, nc - 1 - c, b, 0)),
            pl.BlockSpec((2, H, H4), lambda b, c: (0, 0, 0)),
        ],
        out_specs=[
            pl.BlockSpec((tc, Bh, H), lambda b, c: (c, b, 0)),
            pl.BlockSpec((tc, Bh, H), lambda b, c, nc=nc: (nc - 1 - c, b, 0)),
        ],
        scratch_shapes=[
            pltpu.VMEM((Bh, H), jnp.float32),
            pltpu.VMEM((Bh, H), jnp.float32),
            pltpu.VMEM((Bh, H), jnp.float32),
            pltpu.VMEM((Bh, H), jnp.float32),
        ],
        compiler_params=pltpu.CompilerParams(
            dimension_semantics=("parallel", "arbitrary")),
    )(g, g, whh)


def _head_body(h_ref, w_ref, b_ref, o_ref, *, H, tt):
    a0 = h_ref[0].reshape(-1, H)
    a1 = h_ref[1].reshape(-1, H)
    acc = jnp.dot(a0, w_ref[0], preferred_element_type=jnp.float32)
    acc = acc + jnp.dot(a1, w_ref[1], preferred_element_type=jnp.float32)
    y = jnp.tanh(acc + b_ref[...])
    y = y.reshape(tt, -1, y.shape[-1])
    for i in range(tt):
        o_ref[:, i, :] = y[i]


def _head(h, w, b, *, tt_cap=8, tn_cap=512):
    _, T, Bp, H = h.shape
    N = w.shape[-1]
    tn = _tile(N, tn_cap, 128)
    tt = _div_tile(T, tt_cap)
    grid = (N // tn, T // tt)
    return pl.pallas_call(
        functools.partial(_head_body, H=H, tt=tt),
        out_shape=jax.ShapeDtypeStruct((Bp, T, N), jnp.float32),
        grid=grid,
        in_specs=[
            pl.BlockSpec((2, tt, Bp, H), lambda n, t: (0, t, 0, 0)),
            pl.BlockSpec((2, H, tn), lambda n, t: (0, 0, n)),
            pl.BlockSpec((1, tn), lambda n, t: (0, n)),
        ],
        out_specs=pl.BlockSpec((Bp, tt, tn), lambda n, t: (0, t, n)),
        compiler_params=pltpu.CompilerParams(
            dimension_semantics=("parallel", "parallel")),
    )(h, w, b)


def _head_bm_body(a0_ref, a1_ref, w0_ref, w1_ref, b_ref, o_ref):
    acc = jnp.dot(a0_ref[...], w0_ref[...], preferred_element_type=jnp.float32)
    acc = acc + jnp.dot(a1_ref[...], w1_ref[...], preferred_element_type=jnp.float32)
    o_ref[...] = jnp.tanh(acc + b_ref[...])


def _head_bm(hf, hb, w0, w1, b, *, tn_cap=512):
    T, Bp, H = hf.shape
    N = w0.shape[-1]
    tn = _tile(N, tn_cap, 128)
    a0 = hf.reshape(T, Bp * H)
    a1 = hb.reshape(T, Bp * H)
    grid = (N // tn, Bp)
    tile_bytes = (2 * 2 * T * H * 2 + 2 * 2 * H * tn * 2 + 2 * tn * 4
                  + 2 * T * tn * 4)
    vmem_limit = int(min(64 * 1024 * 1024, max(16 * 1024 * 1024, 2 * tile_bytes)))
    return pl.pallas_call(
        _head_bm_body,
        out_shape=jax.ShapeDtypeStruct((Bp, T, N), jnp.float32),
        grid=grid,
        in_specs=[
            pl.BlockSpec((T, H), lambda n, b: (0, b)),
            pl.BlockSpec((T, H), lambda n, b: (0, b)),
            pl.BlockSpec((None, H, tn), lambda n, b: (0, 0, n)),
            pl.BlockSpec((None, H, tn), lambda n, b: (0, 0, n)),
            pl.BlockSpec((None, 1, tn), lambda n, b: (0, 0, n)),
        ],
        out_specs=pl.BlockSpec((None, T, tn), lambda n, b: (b, 0, n)),
        compiler_params=pltpu.CompilerParams(
            dimension_semantics=("parallel", "parallel"),
            vmem_limit_bytes=vmem_limit),
    )(a0, a1, w0, w1, b)


def _head_tm_body(a0_ref, a1_ref, w0_ref, w1_ref, b_ref, o_ref):
    acc = jnp.dot(a0_ref[...], w0_ref[...], preferred_element_type=jnp.float32)
    acc = acc + jnp.dot(a1_ref[...], w1_ref[...], preferred_element_type=jnp.float32)
    o_ref[...] = jnp.tanh(acc + b_ref[...])


def _head_tm(a0, a1, w0, w1, b, *, tm_cap=512, tn_cap=512):
    M, H = a0.shape
    N = w0.shape[-1]
    tm = _tile(M, tm_cap, 8)
    tn = _tile(N, tn_cap, 128)
    grid = (1, N // tn, M // tm)
    tile_bytes = (2 * 2 * tm * H * 2 + 2 * 2 * H * tn * 2 + 2 * tn * 4
                  + 2 * tm * tn * 4)
    vmem_limit = int(min(64 * 1024 * 1024, max(16 * 1024 * 1024, 2 * tile_bytes)))
    return pl.pallas_call(
        _head_tm_body,
        out_shape=jax.ShapeDtypeStruct((1, M, N), jnp.float32),
        grid=grid,
        in_specs=[
            pl.BlockSpec((tm, H), lambda g, n, m: (m, 0)),
            pl.BlockSpec((tm, H), lambda g, n, m: (m, 0)),
            pl.BlockSpec((None, H, tn), lambda g, n, m: (g, 0, n)),
            pl.BlockSpec((None, H, tn), lambda g, n, m: (g, 0, n)),
            pl.BlockSpec((None, 1, tn), lambda g, n, m: (g, 0, n)),
        ],
        out_specs=pl.BlockSpec((None, tm, tn), lambda g, n, m: (g, m, n)),
        compiler_params=pltpu.CompilerParams(
            dimension_semantics=("parallel", "parallel", "parallel"),
            vmem_limit_bytes=vmem_limit),
    )(a0, a1, w0, w1, b)


def kernel(x, l0_fwd_wih, l0_fwd_whh, l0_fwd_b, l0_bwd_wih, l0_bwd_whh, l0_bwd_b,
           l1_fwd_wih, l1_fwd_whh, l1_fwd_b, l1_bwd_wih, l1_bwd_whh, l1_bwd_b,
           lin_w, lin_b):
    B, T, F = x.shape
    H = l0_fwd_whh.shape[0]
    N = lin_w.shape[1]
    D = N // F
    Bp = _ceil_to(B, 8)
    bf = jnp.bfloat16
    perm = functools.partial(_permute_gates, H=H)

    xt = jnp.transpose(x, (1, 0, 2))
    if Bp != B:
        xt = jnp.pad(xt, ((0, 0), (0, Bp - B), (0, 0)))

    w0 = jnp.stack([perm(l0_fwd_wih), perm(l0_bwd_wih)]).astype(bf)
    b0 = jnp.stack([perm(l0_fwd_b), perm(l0_bwd_b)])
    r0 = jnp.stack([perm(l0_fwd_whh), perm(l0_bwd_whh)]).astype(bf)
    h0f, h0b = _bilstm0(xt, w0, b0, r0)

    w1 = jnp.stack([perm(l1_fwd_wih), perm(l1_bwd_wih)]).astype(bf)
    b1 = jnp.stack([perm(l1_fwd_b), perm(l1_bwd_b)])
    r1 = jnp.stack([perm(l1_fwd_whh), perm(l1_bwd_whh)]).astype(bf)
    h1f, h1b = _bilstm1(h0f, h0b, w1, b1, r1)

    Np = _ceil_to(N, 128)
    lw, lb = lin_w, lin_b
    if Np != N:
        lw = jnp.pad(lw, ((0, 0), (0, Np - N)))
        lb = jnp.pad(lb, ((0, 0), (0, Np - N)))
    y = _head_tm(h1f.reshape(T * Bp, H), h1b.reshape(T * Bp, H),
                 lw[:H][None].astype(bf), lw[H:][None].astype(bf), lb[None])
    y = y[0][:, :N].reshape(T, Bp, N)
    y = jnp.transpose(y, (1, 0, 2))[:B].reshape(B, T * F, D)
    return y


# speedup vs baseline: 1.4610x; 1.0069x over previous
import functools

import jax
import jax.numpy as jnp
from jax.experimental import pallas as pl
from jax.experimental.pallas import tpu as pltpu


def _ceil_to(x, m):
    return (x + m - 1) // m * m


def _tile(dim, cap, align):
    if dim <= cap:
        return dim
    t = (cap // align) * align
    while t > align and dim % t:
        t -= align
    assert dim % t == 0, (dim, cap, align)
    return t


def _div_tile(dim, cap):
    for t in range(min(dim, cap), 0, -1):
        if dim % t == 0:
            return t
    return 1


def _permute_gates(w, H):
    return jnp.concatenate([w[..., :2 * H], w[..., 3 * H:], w[..., 2 * H:3 * H]],
                           axis=-1)


def _gates_body(*refs, n_in):
    a_refs = refs[:n_in]
    w_refs = refs[n_in:2 * n_in]
    b_ref = refs[2 * n_in]
    o_ref = refs[2 * n_in + 1]
    acc = jnp.dot(a_refs[0][...].astype(jnp.bfloat16), w_refs[0][...],
                  preferred_element_type=jnp.float32)
    for i in range(1, n_in):
        acc = acc + jnp.dot(a_refs[i][...].astype(jnp.bfloat16), w_refs[i][...],
                            preferred_element_type=jnp.float32)
    o_ref[...] = (acc + b_ref[...]).astype(o_ref.dtype)


def _input_gates(a_list, w_list, b, *, tm_cap=512, tn_cap=512):
    n_in = len(a_list)
    M = a_list[0].shape[0]
    G, _, N = w_list[0].shape
    tm = _tile(M, tm_cap, 8)
    tn = _tile(N, tn_cap, 128)
    grid = (G, N // tn, M // tm)
    in_specs = []
    for a in a_list:
        in_specs.append(pl.BlockSpec((tm, a.shape[1]), lambda g, n, m: (m, 0)))
    for w in w_list:
        in_specs.append(pl.BlockSpec((None, w.shape[1], tn),
                                     lambda g, n, m: (g, 0, n)))
    in_specs.append(pl.BlockSpec((None, 1, tn), lambda g, n, m: (g, 0, n)))
    tile_bytes = (sum(2 * tm * a.shape[1] * a.dtype.itemsize for a in a_list)
                  + sum(2 * w.shape[1] * tn * 2 for w in w_list)
                  + 2 * tn * 4 + 2 * tm * tn * 2)
    vmem_limit = int(min(64 * 1024 * 1024, max(16 * 1024 * 1024, 2 * tile_bytes)))
    return pl.pallas_call(
        functools.partial(_gates_body, n_in=n_in),
        out_shape=jax.ShapeDtypeStruct((G, M, N), jnp.bfloat16),
        grid=grid,
        in_specs=in_specs,
        out_specs=pl.BlockSpec((None, tm, tn), lambda g, n, m: (g, m, n)),
        compiler_params=pltpu.CompilerParams(
            dimension_semantics=("parallel", "parallel", "parallel"),
            vmem_limit_bytes=vmem_limit),
    )(*a_list, *w_list, b)


def _lstm0_body(xf_ref, xb_ref, wih_ref, bias_ref, whh_ref, hf_ref, hb_ref,
                hf_sc, cf_sc, hb_sc, cb_sc, *, H, tc):
    @pl.when(pl.program_id(1) == 0)
    def _():
        hf_sc[...] = jnp.zeros_like(hf_sc)
        cf_sc[...] = jnp.zeros_like(cf_sc)
        hb_sc[...] = jnp.zeros_like(hb_sc)
        cb_sc[...] = jnp.zeros_like(cb_sc)

    bf16 = jnp.bfloat16
    Bh = xf_ref.shape[1]
    F = xf_ref.shape[2]
    H4 = 4 * H
    gf = (jnp.dot(xf_ref[...].reshape(tc * Bh, F).astype(bf16), wih_ref[0],
                  preferred_element_type=jnp.float32)
          + bias_ref[0]).astype(bf16).reshape(tc, Bh, H4)
    gb = (jnp.dot(xb_ref[...].reshape(tc * Bh, F).astype(bf16), wih_ref[1],
                  preferred_element_type=jnp.float32)
          + bias_ref[1]).astype(bf16).reshape(tc, Bh, H4)

    wf = whh_ref[0]
    wb = whh_ref[1]
    hf, cf = hf_sc[...], cf_sc[...]
    hb, cb = hb_sc[...], cb_sc[...]
    for t in range(tc):
        tb = tc - 1 - t
        zf = gf[t].astype(jnp.float32) + jnp.dot(
            hf.astype(bf16), wf, preferred_element_type=jnp.float32)
        zb = gb[tb].astype(jnp.float32) + jnp.dot(
            hb.astype(bf16), wb, preferred_element_type=jnp.float32)
        pf = jax.nn.sigmoid(zf[:, :3 * H])
        pb = jax.nn.sigmoid(zb[:, :3 * H])
        cf = pf[:, H:2 * H] * cf + pf[:, :H] * jnp.tanh(zf[:, 3 * H:])
        cb = pb[:, H:2 * H] * cb + pb[:, :H] * jnp.tanh(zb[:, 3 * H:])
        hf = pf[:, 2 * H:] * jnp.tanh(cf)
        hb = pb[:, 2 * H:] * jnp.tanh(cb)
        hf_ref[t] = hf.astype(bf16)
        hb_ref[tb] = hb.astype(bf16)
    hf_sc[...], cf_sc[...] = hf, cf
    hb_sc[...], cb_sc[...] = hb, cb


def _bilstm0(x_tbf, wih, bias, whh, *, tc_cap=32):
    T, Bp, F = x_tbf.shape
    H4 = whh.shape[-1]
    H = H4 // 4
    tc = _div_tile(T, tc_cap)
    nc = T // tc
    nb = 2 if Bp % 16 == 0 else 1
    Bh = Bp // nb

    out_shape = [jax.ShapeDtypeStruct((T, Bp, H), jnp.bfloat16)] * 2
    return pl.pallas_call(
        functools.partial(_lstm0_body, H=H, tc=tc),
        out_shape=out_shape,
        grid=(nb, nc),
        in_specs=[
            pl.BlockSpec((tc, Bh, F), lambda b, c: (c, b, 0)),
            pl.BlockSpec((tc, Bh, F), lambda b, c, nc=nc: (nc - 1 - c, b, 0)),
            pl.BlockSpec((2, F, H4), lambda b, c: (0, 0, 0)),
            pl.BlockSpec((2, 1, H4), lambda b, c: (0, 0, 0)),
            pl.BlockSpec((2, H, H4), lambda b, c: (0, 0, 0)),
        ],
        out_specs=[
            pl.BlockSpec((tc, Bh, H), lambda b, c: (c, b, 0)),
            pl.BlockSpec((tc, Bh, H), lambda b, c, nc=nc: (nc - 1 - c, b, 0)),
        ],
        scratch_shapes=[
            pltpu.VMEM((Bh, H), jnp.float32),
            pltpu.VMEM((Bh, H), jnp.float32),
            pltpu.VMEM((Bh, H), jnp.float32),
            pltpu.VMEM((Bh, H), jnp.float32),
        ],
        compiler_params=pltpu.CompilerParams(
            dimension_semantics=("parallel", "arbitrary")),
    )(x_tbf, x_tbf, wih, bias, whh)


def _lstm1_body(af_ref, bf_ref, ab_ref, bb_ref, wih_ref, bias_ref, whh_ref,
                hf_ref, hb_ref, hf_sc, cf_sc, hb_sc, cb_sc, *, H, tc):
    @pl.when(pl.program_id(1) == 0)
    def _():
        hf_sc[...] = jnp.zeros_like(hf_sc)
        cf_sc[...] = jnp.zeros_like(cf_sc)
        hb_sc[...] = jnp.zeros_like(hb_sc)
        cb_sc[...] = jnp.zeros_like(cb_sc)

    bf16 = jnp.bfloat16
    Bh = af_ref.shape[1]
    H4 = 4 * H
    gf = (jnp.dot(af_ref[...].reshape(tc * Bh, H), wih_ref[0, :H],
                  preferred_element_type=jnp.float32)
          + jnp.dot(bf_ref[...].reshape(tc * Bh, H), wih_ref[0, H:],
                    preferred_element_type=jnp.float32)
          + bias_ref[0]).astype(bf16).reshape(tc, Bh, H4)
    gb = (jnp.dot(ab_ref[...].reshape(tc * Bh, H), wih_ref[1, :H],
                  preferred_element_type=jnp.float32)
          + jnp.dot(bb_ref[...].reshape(tc * Bh, H), wih_ref[1, H:],
                    preferred_element_type=jnp.float32)
          + bias_ref[1]).astype(bf16).reshape(tc, Bh, H4)

    wf = whh_ref[0]
    wb = whh_ref[1]
    hf, cf = hf_sc[...], cf_sc[...]
    hb, cb = hb_sc[...], cb_sc[...]
    for t in range(tc):
        tb = tc - 1 - t
        zf = gf[t].astype(jnp.float32) + jnp.dot(
            hf.astype(bf16), wf, preferred_element_type=jnp.float32)
        zb = gb[tb].astype(jnp.float32) + jnp.dot(
            hb.astype(bf16), wb, preferred_element_type=jnp.float32)
        pf = jax.nn.sigmoid(zf[:, :3 * H])
        pb = jax.nn.sigmoid(zb[:, :3 * H])
        cf = pf[:, H:2 * H] * cf + pf[:, :H] * jnp.tanh(zf[:, 3 * H:])
        cb = pb[:, H:2 * H] * cb + pb[:, :H] * jnp.tanh(zb[:, 3 * H:])
        hf = pf[:, 2 * H:] * jnp.tanh(cf)
        hb = pb[:, 2 * H:] * jnp.tanh(cb)
        hf_ref[t] = hf.astype(bf16)
        hb_ref[tb] = hb.astype(bf16)
    hf_sc[...], cf_sc[...] = hf, cf
    hb_sc[...], cb_sc[...] = hb, cb


def _bilstm1(h0f, h0b, wih, bias, whh, *, tc_cap=32):
    T, Bp, H = h0f.shape
    H4 = whh.shape[-1]
    tc = _div_tile(T, tc_cap)
    nc = T // tc
    nb = 2 if Bp % 16 == 0 else 1
    Bh = Bp // nb

    fwd = lambda b, c: (c, b, 0)
    bwd = lambda b, c, nc=nc: (nc - 1 - c, b, 0)
    out_shape = [jax.ShapeDtypeStruct((T, Bp, H), jnp.bfloat16)] * 2
    return pl.pallas_call(
        functools.partial(_lstm1_body, H=H, tc=tc),
        out_shape=out_shape,
        grid=(nb, nc),
        in_specs=[
            pl.BlockSpec((tc, Bh, H), fwd),
            pl.BlockSpec((tc, Bh, H), fwd),
            pl.BlockSpec((tc, Bh, H), bwd),
            pl.BlockSpec((tc, Bh, H), bwd),
            pl.BlockSpec((2, 2 * H, H4), lambda b, c: (0, 0, 0)),
            pl.BlockSpec((2, 1, H4), lambda b, c: (0, 0, 0)),
            pl.BlockSpec((2, H, H4), lambda b, c: (0, 0, 0)),
        ],
        out_specs=[
            pl.BlockSpec((tc, Bh, H), fwd),
            pl.BlockSpec((tc, Bh, H), bwd),
        ],
        scratch_shapes=[
            pltpu.VMEM((Bh, H), jnp.float32),
            pltpu.VMEM((Bh, H), jnp.float32),
            pltpu.VMEM((Bh, H), jnp.float32),
            pltpu.VMEM((Bh, H), jnp.float32),
        ],
        compiler_params=pltpu.CompilerParams(
            dimension_semantics=("parallel", "arbitrary")),
    )(h0f, h0b, h0f, h0b, wih, bias, whh)


def _lstm_body(gf_ref, gb_ref, whh_ref, hf_ref, hb_ref,
               hf_sc, cf_sc, hb_sc, cb_sc, *, H, tc):
    @pl.when(pl.program_id(1) == 0)
    def _():
        hf_sc[...] = jnp.zeros_like(hf_sc)
        cf_sc[...] = jnp.zeros_like(cf_sc)
        hb_sc[...] = jnp.zeros_like(hb_sc)
        cb_sc[...] = jnp.zeros_like(cb_sc)

    wf = whh_ref[0]
    wb = whh_ref[1]
    hf, cf = hf_sc[...], cf_sc[...]
    hb, cb = hb_sc[...], cb_sc[...]
    bf16 = jnp.bfloat16
    for t in range(tc):
        tb = tc - 1 - t
        zf = gf_ref[t].astype(jnp.float32) + jnp.dot(
            hf.astype(bf16), wf, preferred_element_type=jnp.float32)
        zb = gb_ref[tb].astype(jnp.float32) + jnp.dot(
            hb.astype(bf16), wb, preferred_element_type=jnp.float32)
        pf = jax.nn.sigmoid(zf[:, :3 * H])
        pb = jax.nn.sigmoid(zb[:, :3 * H])
        cf = pf[:, H:2 * H] * cf + pf[:, :H] * jnp.tanh(zf[:, 3 * H:])
        cb = pb[:, H:2 * H] * cb + pb[:, :H] * jnp.tanh(zb[:, 3 * H:])
        hf = pf[:, 2 * H:] * jnp.tanh(cf)
        hb = pb[:, 2 * H:] * jnp.tanh(cb)
        hf_ref[t] = hf.astype(bf16)
        hb_ref[tb] = hb.astype(bf16)
    hf_sc[...], cf_sc[...] = hf, cf
    hb_sc[...], cb_sc[...] = hb, cb


def _bilstm(g, whh, *, tc_cap=32):
    _, T, Bp, H4 = g.shape
    H = H4 // 4
    tc = _div_tile(T, tc_cap)
    nc = T // tc
    nb = 2 if Bp % 16 == 0 else 1
    Bh = Bp // nb

    out_shape = [jax.ShapeDtypeStruct((T, Bp, H), jnp.bfloat16)] * 2
    return pl.pallas_call(
        functools.partial(_lstm_body, H=H, tc=tc),
        out_shape=out_shape,
        grid=(nb, nc),
        in_specs=[
            pl.BlockSpec((None, tc, Bh, H4), lambda b, c: (0, c, b, 0)),
            pl.BlockSpec((None, tc, Bh, H4),
                         lambda b, c, nc=nc: (1, nc - 1 - c, b, 0)),
            pl.BlockSpec((2, H, H4), lambda b, c: (0, 0, 0)),
        ],
        out_specs=[
            pl.BlockSpec((tc, Bh, H), lambda b, c: (c, b, 0)),
            pl.BlockSpec((tc, Bh, H), lambda b, c, nc=nc: (nc - 1 - c, b, 0)),
        ],
        scratch_shapes=[
            pltpu.VMEM((Bh, H), jnp.float32),
            pltpu.VMEM((Bh, H), jnp.float32),
            pltpu.VMEM((Bh, H), jnp.float32),
            pltpu.VMEM((Bh, H), jnp.float32),
        ],
        compiler_params=pltpu.CompilerParams(
            dimension_semantics=("parallel", "arbitrary")),
    )(g, g, whh)


def _head_body(h_ref, w_ref, b_ref, o_ref, *, H, tt):
    a0 = h_ref[0].reshape(-1, H)
    a1 = h_ref[1].reshape(-1, H)
    acc = jnp.dot(a0, w_ref[0], preferred_element_type=jnp.float32)
    acc = acc + jnp.dot(a1, w_ref[1], preferred_element_type=jnp.float32)
    y = jnp.tanh(acc + b_ref[...])
    y = y.reshape(tt, -1, y.shape[-1])
    for i in range(tt):
        o_ref[:, i, :] = y[i]


def _head(h, w, b, *, tt_cap=8, tn_cap=512):
    _, T, Bp, H = h.shape
    N = w.shape[-1]
    tn = _tile(N, tn_cap, 128)
    tt = _div_tile(T, tt_cap)
    grid = (N // tn, T // tt)
    return pl.pallas_call(
        functools.partial(_head_body, H=H, tt=tt),
        out_shape=jax.ShapeDtypeStruct((Bp, T, N), jnp.float32),
        grid=grid,
        in_specs=[
            pl.BlockSpec((2, tt, Bp, H), lambda n, t: (0, t, 0, 0)),
            pl.BlockSpec((2, H, tn), lambda n, t: (0, 0, n)),
            pl.BlockSpec((1, tn), lambda n, t: (0, n)),
        ],
        out_specs=pl.BlockSpec((Bp, tt, tn), lambda n, t: (0, t, n)),
        compiler_params=pltpu.CompilerParams(
            dimension_semantics=("parallel", "parallel")),
    )(h, w, b)


def _head_bm_body(a0_ref, a1_ref, w0_ref, w1_ref, b_ref, o_ref):
    acc = jnp.dot(a0_ref[...], w0_ref[...], preferred_element_type=jnp.float32)
    acc = acc + jnp.dot(a1_ref[...], w1_ref[...], preferred_element_type=jnp.float32)
    o_ref[...] = jnp.tanh(acc + b_ref[...])


def _head_bm(hf, hb, w0, w1, b, *, tn_cap=512):
    T, Bp, H = hf.shape
    N = w0.shape[-1]
    tn = _tile(N, tn_cap, 128)
    a0 = hf.reshape(T, Bp * H)
    a1 = hb.reshape(T, Bp * H)
    grid = (N // tn, Bp)
    tile_bytes = (2 * 2 * T * H * 2 + 2 * 2 * H * tn * 2 + 2 * tn * 4
                  + 2 * T * tn * 4)
    vmem_limit = int(min(64 * 1024 * 1024, max(16 * 1024 * 1024, 2 * tile_bytes)))
    return pl.pallas_call(
        _head_bm_body,
        out_shape=jax.ShapeDtypeStruct((Bp, T, N), jnp.float32),
        grid=grid,
        in_specs=[
            pl.BlockSpec((T, H), lambda n, b: (0, b)),
            pl.BlockSpec((T, H), lambda n, b: (0, b)),
            pl.BlockSpec((None, H, tn), lambda n, b: (0, 0, n)),
            pl.BlockSpec((None, H, tn), lambda n, b: (0, 0, n)),
            pl.BlockSpec((None, 1, tn), lambda n, b: (0, 0, n)),
        ],
        out_specs=pl.BlockSpec((None, T, tn), lambda n, b: (b, 0, n)),
        compiler_params=pltpu.CompilerParams(
            dimension_semantics=("parallel", "parallel"),
            vmem_limit_bytes=vmem_limit),
    )(a0, a1, w0, w1, b)


def _head_tm_body(a0_ref, a1_ref, w0_ref, w1_ref, b_ref, o_ref):
    acc = jnp.dot(a0_ref[...], w0_ref[...], preferred_element_type=jnp.float32)
    acc = acc + jnp.dot(a1_ref[...], w1_ref[...], preferred_element_type=jnp.float32)
    o_ref[...] = jnp.tanh(acc + b_ref[...])


def _head_tm(a0, a1, w0, w1, b, *, tm_cap=512, tn_cap=512):
    M, H = a0.shape
    N = w0.shape[-1]
    tm = _tile(M, tm_cap, 8)
    tn = _tile(N, tn_cap, 128)
    grid = (1, N // tn, M // tm)
    tile_bytes = (2 * 2 * tm * H * 2 + 2 * 2 * H * tn * 2 + 2 * tn * 4
                  + 2 * tm * tn * 4)
    vmem_limit = int(min(64 * 1024 * 1024, max(16 * 1024 * 1024, 2 * tile_bytes)))
    return pl.pallas_call(
        _head_tm_body,
        out_shape=jax.ShapeDtypeStruct((1, M, N), jnp.float32),
        grid=grid,
        in_specs=[
            pl.BlockSpec((tm, H), lambda g, n, m: (m, 0)),
            pl.BlockSpec((tm, H), lambda g, n, m: (m, 0)),
            pl.BlockSpec((None, H, tn), lambda g, n, m: (g, 0, n)),
            pl.BlockSpec((None, H, tn), lambda g, n, m: (g, 0, n)),
            pl.BlockSpec((None, 1, tn), lambda g, n, m: (g, 0, n)),
        ],
        out_specs=pl.BlockSpec((None, tm, tn), lambda g, n, m: (g, m, n)),
        compiler_params=pltpu.CompilerParams(
            dimension_semantics=("parallel", "parallel", "parallel"),
            vmem_limit_bytes=vmem_limit),
    )(a0, a1, w0, w1, b)


def kernel(x, l0_fwd_wih, l0_fwd_whh, l0_fwd_b, l0_bwd_wih, l0_bwd_whh, l0_bwd_b,
           l1_fwd_wih, l1_fwd_whh, l1_fwd_b, l1_bwd_wih, l1_bwd_whh, l1_bwd_b,
           lin_w, lin_b):
    B, T, F = x.shape
    H = l0_fwd_whh.shape[0]
    N = lin_w.shape[1]
    D = N // F
    Bp = _ceil_to(B, 8)
    bf = jnp.bfloat16
    perm = functools.partial(_permute_gates, H=H)

    xt = jnp.transpose(x, (1, 0, 2))
    if Bp != B:
        xt = jnp.pad(xt, ((0, 0), (0, Bp - B), (0, 0)))

    w0 = jnp.stack([perm(l0_fwd_wih), perm(l0_bwd_wih)]).astype(bf)
    b0 = jnp.stack([perm(l0_fwd_b), perm(l0_bwd_b)])
    r0 = jnp.stack([perm(l0_fwd_whh), perm(l0_bwd_whh)]).astype(bf)
    h0f, h0b = _bilstm0(xt, w0, b0, r0)

    w1 = jnp.stack([perm(l1_fwd_wih), perm(l1_bwd_wih)]).astype(bf)
    b1 = jnp.stack([perm(l1_fwd_b), perm(l1_bwd_b)])
    r1 = jnp.stack([perm(l1_fwd_whh), perm(l1_bwd_whh)]).astype(bf)
    h1f, h1b = _bilstm1(h0f, h0b, w1, b1, r1)

    Np = _ceil_to(N, 128)
    lw, lb = lin_w, lin_b
    if Np != N:
        lw = jnp.pad(lw, ((0, 0), (0, Np - N)))
        lb = jnp.pad(lb, ((0, 0), (0, Np - N)))
    y = _head_tm(h1f.reshape(T * Bp, H), h1b.reshape(T * Bp, H),
                 lw[:H][None].astype(bf), lw[H:][None].astype(bf), lb[None])
    y = y[0][:, :N].reshape(T, Bp, N)
    y = jnp.transpose(y, (1, 0, 2))[:B].reshape(B, T * F, D)
    return y


# speedup vs baseline: 1.5637x; 1.0703x over previous
import functools

import jax
import jax.numpy as jnp
from jax.experimental import pallas as pl
from jax.experimental.pallas import tpu as pltpu


def _ceil_to(x, m):
    return (x + m - 1) // m * m


def _tile(dim, cap, align):
    if dim <= cap:
        return dim
    t = (cap // align) * align
    while t > align and dim % t:
        t -= align
    assert dim % t == 0, (dim, cap, align)
    return t


def _div_tile(dim, cap):
    for t in range(min(dim, cap), 0, -1):
        if dim % t == 0:
            return t
    return 1


def _permute_gates(w, H):
    return jnp.concatenate([w[..., :2 * H], w[..., 3 * H:], w[..., 2 * H:3 * H]],
                           axis=-1)


def _gates_body(*refs, n_in):
    a_refs = refs[:n_in]
    w_refs = refs[n_in:2 * n_in]
    b_ref = refs[2 * n_in]
    o_ref = refs[2 * n_in + 1]
    acc = jnp.dot(a_refs[0][...].astype(jnp.bfloat16), w_refs[0][...],
                  preferred_element_type=jnp.float32)
    for i in range(1, n_in):
        acc = acc + jnp.dot(a_refs[i][...].astype(jnp.bfloat16), w_refs[i][...],
                            preferred_element_type=jnp.float32)
    o_ref[...] = (acc + b_ref[...]).astype(o_ref.dtype)


def _input_gates(a_list, w_list, b, *, tm_cap=512, tn_cap=512):
    n_in = len(a_list)
    M = a_list[0].shape[0]
    G, _, N = w_list[0].shape
    tm = _tile(M, tm_cap, 8)
    tn = _tile(N, tn_cap, 128)
    grid = (G, N // tn, M // tm)
    in_specs = []
    for a in a_list:
        in_specs.append(pl.BlockSpec((tm, a.shape[1]), lambda g, n, m: (m, 0)))
    for w in w_list:
        in_specs.append(pl.BlockSpec((None, w.shape[1], tn),
                                     lambda g, n, m: (g, 0, n)))
    in_specs.append(pl.BlockSpec((None, 1, tn), lambda g, n, m: (g, 0, n)))
    tile_bytes = (sum(2 * tm * a.shape[1] * a.dtype.itemsize for a in a_list)
                  + sum(2 * w.shape[1] * tn * 2 for w in w_list)
                  + 2 * tn * 4 + 2 * tm * tn * 2)
    vmem_limit = int(min(64 * 1024 * 1024, max(16 * 1024 * 1024, 2 * tile_bytes)))
    return pl.pallas_call(
        functools.partial(_gates_body, n_in=n_in),
        out_shape=jax.ShapeDtypeStruct((G, M, N), jnp.bfloat16),
        grid=grid,
        in_specs=in_specs,
        out_specs=pl.BlockSpec((None, tm, tn), lambda g, n, m: (g, m, n)),
        compiler_params=pltpu.CompilerParams(
            dimension_semantics=("parallel", "parallel", "parallel"),
            vmem_limit_bytes=vmem_limit),
    )(*a_list, *w_list, b)


def _lstm0_body(xf_ref, xb_ref, wih_ref, bias_ref, whh_ref, hf_ref, hb_ref,
                hf_sc, cf_sc, hb_sc, cb_sc, *, H, tc):
    @pl.when(pl.program_id(1) == 0)
    def _():
        hf_sc[...] = jnp.zeros_like(hf_sc)
        cf_sc[...] = jnp.zeros_like(cf_sc)
        hb_sc[...] = jnp.zeros_like(hb_sc)
        cb_sc[...] = jnp.zeros_like(cb_sc)

    bf16 = jnp.bfloat16
    Bh = xf_ref.shape[1]
    F = xf_ref.shape[2]
    H4 = 4 * H
    gf = (jnp.dot(xf_ref[...].reshape(tc * Bh, F).astype(bf16), wih_ref[0],
                  preferred_element_type=jnp.float32)
          + bias_ref[0]).astype(bf16).reshape(tc, Bh, H4)
    gb = (jnp.dot(xb_ref[...].reshape(tc * Bh, F).astype(bf16), wih_ref[1],
                  preferred_element_type=jnp.float32)
          + bias_ref[1]).astype(bf16).reshape(tc, Bh, H4)

    wf = whh_ref[0]
    wb = whh_ref[1]
    hf, cf = hf_sc[...], cf_sc[...]
    hb, cb = hb_sc[...], cb_sc[...]
    for t in range(tc):
        tb = tc - 1 - t
        zf = gf[t].astype(jnp.float32) + jnp.dot(
            hf.astype(bf16), wf, preferred_element_type=jnp.float32)
        zb = gb[tb].astype(jnp.float32) + jnp.dot(
            hb.astype(bf16), wb, preferred_element_type=jnp.float32)
        pf = jax.nn.sigmoid(zf[:, :3 * H])
        pb = jax.nn.sigmoid(zb[:, :3 * H])
        cf = pf[:, H:2 * H] * cf + pf[:, :H] * jnp.tanh(zf[:, 3 * H:])
        cb = pb[:, H:2 * H] * cb + pb[:, :H] * jnp.tanh(zb[:, 3 * H:])
        hf = pf[:, 2 * H:] * jnp.tanh(cf)
        hb = pb[:, 2 * H:] * jnp.tanh(cb)
        hf_ref[t] = hf.astype(bf16)
        hb_ref[tb] = hb.astype(bf16)
    hf_sc[...], cf_sc[...] = hf, cf
    hb_sc[...], cb_sc[...] = hb, cb


def _bilstm0(x_tbf, wih, bias, whh, *, tc_cap=32):
    T, Bp, F = x_tbf.shape
    H4 = whh.shape[-1]
    H = H4 // 4
    tc = _div_tile(T, tc_cap)
    nc = T // tc
    nb = 2 if Bp % 16 == 0 else 1
    Bh = Bp // nb

    out_shape = [jax.ShapeDtypeStruct((T, Bp, H), jnp.bfloat16)] * 2
    return pl.pallas_call(
        functools.partial(_lstm0_body, H=H, tc=tc),
        out_shape=out_shape,
        grid=(nb, nc),
        in_specs=[
            pl.BlockSpec((tc, Bh, F), lambda b, c: (c, b, 0)),
            pl.BlockSpec((tc, Bh, F), lambda b, c, nc=nc: (nc - 1 - c, b, 0)),
            pl.BlockSpec((2, F, H4), lambda b, c: (0, 0, 0)),
            pl.BlockSpec((2, 1, H4), lambda b, c: (0, 0, 0)),
            pl.BlockSpec((2, H, H4), lambda b, c: (0, 0, 0)),
        ],
        out_specs=[
            pl.BlockSpec((tc, Bh, H), lambda b, c: (c, b, 0)),
            pl.BlockSpec((tc, Bh, H), lambda b, c, nc=nc: (nc - 1 - c, b, 0)),
        ],
        scratch_shapes=[
            pltpu.VMEM((Bh, H), jnp.float32),
            pltpu.VMEM((Bh, H), jnp.float32),
            pltpu.VMEM((Bh, H), jnp.float32),
            pltpu.VMEM((Bh, H), jnp.float32),
        ],
        compiler_params=pltpu.CompilerParams(
            dimension_semantics=("parallel", "arbitrary")),
    )(x_tbf, x_tbf, wih, bias, whh)


def _lstm1_body(af_ref, bf_ref, ab_ref, bb_ref, wih_ref, bias_ref, whh_ref,
                hf_ref, hb_ref, hf_sc, cf_sc, hb_sc, cb_sc, *, H, tc):
    @pl.when(pl.program_id(1) == 0)
    def _():
        hf_sc[...] = jnp.zeros_like(hf_sc)
        cf_sc[...] = jnp.zeros_like(cf_sc)
        hb_sc[...] = jnp.zeros_like(hb_sc)
        cb_sc[...] = jnp.zeros_like(cb_sc)

    bf16 = jnp.bfloat16
    Bh = af_ref.shape[1]
    H4 = 4 * H
    gf = (jnp.dot(af_ref[...].reshape(tc * Bh, H), wih_ref[0, :H],
                  preferred_element_type=jnp.float32)
          + jnp.dot(bf_ref[...].reshape(tc * Bh, H), wih_ref[0, H:],
                    preferred_element_type=jnp.float32)
          + bias_ref[0]).astype(bf16).reshape(tc, Bh, H4)
    gb = (jnp.dot(ab_ref[...].reshape(tc * Bh, H), wih_ref[1, :H],
                  preferred_element_type=jnp.float32)
          + jnp.dot(bb_ref[...].reshape(tc * Bh, H), wih_ref[1, H:],
                    preferred_element_type=jnp.float32)
          + bias_ref[1]).astype(bf16).reshape(tc, Bh, H4)

    wf = whh_ref[0]
    wb = whh_ref[1]
    hf, cf = hf_sc[...], cf_sc[...]
    hb, cb = hb_sc[...], cb_sc[...]
    for t in range(tc):
        tb = tc - 1 - t
        zf = gf[t].astype(jnp.float32) + jnp.dot(
            hf.astype(bf16), wf, preferred_element_type=jnp.float32)
        zb = gb[tb].astype(jnp.float32) + jnp.dot(
            hb.astype(bf16), wb, preferred_element_type=jnp.float32)
        pf = jax.nn.sigmoid(zf[:, :3 * H])
        pb = jax.nn.sigmoid(zb[:, :3 * H])
        cf = pf[:, H:2 * H] * cf + pf[:, :H] * jnp.tanh(zf[:, 3 * H:])
        cb = pb[:, H:2 * H] * cb + pb[:, :H] * jnp.tanh(zb[:, 3 * H:])
        hf = pf[:, 2 * H:] * jnp.tanh(cf)
        hb = pb[:, 2 * H:] * jnp.tanh(cb)
        hf_ref[t] = hf.astype(bf16)
        hb_ref[tb] = hb.astype(bf16)
    hf_sc[...], cf_sc[...] = hf, cf
    hb_sc[...], cb_sc[...] = hb, cb


def _bilstm1(h0f, h0b, wih, bias, whh, *, tc_cap=32):
    T, Bp, H = h0f.shape
    H4 = whh.shape[-1]
    tc = _div_tile(T, tc_cap)
    nc = T // tc
    nb = 2 if Bp % 16 == 0 else 1
    Bh = Bp // nb

    fwd = lambda b, c: (c, b, 0)
    bwd = lambda b, c, nc=nc: (nc - 1 - c, b, 0)
    out_shape = [jax.ShapeDtypeStruct((T, Bp, H), jnp.bfloat16)] * 2
    return pl.pallas_call(
        functools.partial(_lstm1_body, H=H, tc=tc),
        out_shape=out_shape,
        grid=(nb, nc),
        in_specs=[
            pl.BlockSpec((tc, Bh, H), fwd),
            pl.BlockSpec((tc, Bh, H), fwd),
            pl.BlockSpec((tc, Bh, H), bwd),
            pl.BlockSpec((tc, Bh, H), bwd),
            pl.BlockSpec((2, 2 * H, H4), lambda b, c: (0, 0, 0)),
            pl.BlockSpec((2, 1, H4), lambda b, c: (0, 0, 0)),
            pl.BlockSpec((2, H, H4), lambda b, c: (0, 0, 0)),
        ],
        out_specs=[
            pl.BlockSpec((tc, Bh, H), fwd),
            pl.BlockSpec((tc, Bh, H), bwd),
        ],
        scratch_shapes=[
            pltpu.VMEM((Bh, H), jnp.float32),
            pltpu.VMEM((Bh, H), jnp.float32),
            pltpu.VMEM((Bh, H), jnp.float32),
            pltpu.VMEM((Bh, H), jnp.float32),
        ],
        compiler_params=pltpu.CompilerParams(
            dimension_semantics=("parallel", "arbitrary")),
    )(h0f, h0b, h0f, h0b, wih, bias, whh)


def _lstm_body(gf_ref, gb_ref, whh_ref, hf_ref, hb_ref,
               hf_sc, cf_sc, hb_sc, cb_sc, *, H, tc):
    @pl.when(pl.program_id(1) == 0)
    def _():
        hf_sc[...] = jnp.zeros_like(hf_sc)
        cf_sc[...] = jnp.zeros_like(cf_sc)
        hb_sc[...] = jnp.zeros_like(hb_sc)
        cb_sc[...] = jnp.zeros_like(cb_sc)

    wf = whh_ref[0]
    wb = whh_ref[1]
    hf, cf = hf_sc[...], cf_sc[...]
    hb, cb = hb_sc[...], cb_sc[...]
    bf16 = jnp.bfloat16
    for t in range(tc):
        tb = tc - 1 - t
        zf = gf_ref[t].astype(jnp.float32) + jnp.dot(
            hf.astype(bf16), wf, preferred_element_type=jnp.float32)
        zb = gb_ref[tb].astype(jnp.float32) + jnp.dot(
            hb.astype(bf16), wb, preferred_element_type=jnp.float32)
        pf = jax.nn.sigmoid(zf[:, :3 * H])
        pb = jax.nn.sigmoid(zb[:, :3 * H])
        cf = pf[:, H:2 * H] * cf + pf[:, :H] * jnp.tanh(zf[:, 3 * H:])
        cb = pb[:, H:2 * H] * cb + pb[:, :H] * jnp.tanh(zb[:, 3 * H:])
        hf = pf[:, 2 * H:] * jnp.tanh(cf)
        hb = pb[:, 2 * H:] * jnp.tanh(cb)
        hf_ref[t] = hf.astype(bf16)
        hb_ref[tb] = hb.astype(bf16)
    hf_sc[...], cf_sc[...] = hf, cf
    hb_sc[...], cb_sc[...] = hb, cb


def _bilstm(g, whh, *, tc_cap=32):
    _, T, Bp, H4 = g.shape
    H = H4 // 4
    tc = _div_tile(T, tc_cap)
    nc = T // tc
    nb = 2 if Bp % 16 == 0 else 1
    Bh = Bp // nb

    out_shape = [jax.ShapeDtypeStruct((T, Bp, H), jnp.bfloat16)] * 2
    return pl.pallas_call(
        functools.partial(_lstm_body, H=H, tc=tc),
        out_shape=out_shape,
        grid=(nb, nc),
        in_specs=[
            pl.BlockSpec((None, tc, Bh, H4), lambda b, c: (0, c, b, 0)),
            pl.BlockSpec((None, tc, Bh, H4),
                         lambda b, c, nc=nc: (1, nc - 1 - c, b, 0)),
            pl.BlockSpec((2, H, H4), lambda b, c: (0, 0, 0)),
        ],
        out_specs=[
            pl.BlockSpec((tc, Bh, H), lambda b, c: (c, b, 0)),
            pl.BlockSpec((tc, Bh, H), lambda b, c, nc=nc: (nc - 1 - c, b, 0)),
        ],
        scratch_shapes=[
            pltpu.VMEM((Bh, H), jnp.float32),
            pltpu.VMEM((Bh, H), jnp.float32),
            pltpu.VMEM((Bh, H), jnp.float32),
            pltpu.VMEM((Bh, H), jnp.float32),
        ],
        compiler_params=pltpu.CompilerParams(
            dimension_semantics=("parallel", "arbitrary")),
    )(g, g, whh)


def _head_body(h_ref, w_ref, b_ref, o_ref, *, H, tt):
    a0 = h_ref[0].reshape(-1, H)
    a1 = h_ref[1].reshape(-1, H)
    acc = jnp.dot(a0, w_ref[0], preferred_element_type=jnp.float32)
    acc = acc + jnp.dot(a1, w_ref[1], preferred_element_type=jnp.float32)
    y = jnp.tanh(acc + b_ref[...])
    y = y.reshape(tt, -1, y.shape[-1])
    for i in range(tt):
        o_ref[:, i, :] = y[i]


def _head(h, w, b, *, tt_cap=8, tn_cap=512):
    _, T, Bp, H = h.shape
    N = w.shape[-1]
    tn = _tile(N, tn_cap, 128)
    tt = _div_tile(T, tt_cap)
    grid = (N // tn, T // tt)
    return pl.pallas_call(
        functools.partial(_head_body, H=H, tt=tt),
        out_shape=jax.ShapeDtypeStruct((Bp, T, N), jnp.float32),
        grid=grid,
        in_specs=[
            pl.BlockSpec((2, tt, Bp, H), lambda n, t: (0, t, 0, 0)),
            pl.BlockSpec((2, H, tn), lambda n, t: (0, 0, n)),
            pl.BlockSpec((1, tn), lambda n, t: (0, n)),
        ],
        out_specs=pl.BlockSpec((Bp, tt, tn), lambda n, t: (0, t, n)),
        compiler_params=pltpu.CompilerParams(
            dimension_semantics=("parallel", "parallel")),
    )(h, w, b)


def _head_bm_body(a0_ref, a1_ref, w0_ref, w1_ref, b_ref, o_ref):
    acc = jnp.dot(a0_ref[...], w0_ref[...], preferred_element_type=jnp.float32)
    acc = acc + jnp.dot(a1_ref[...], w1_ref[...], preferred_element_type=jnp.float32)
    o_ref[...] = jnp.tanh(acc + b_ref[...])


def _head_bm(hf, hb, w0, w1, b, *, tn_cap=512):
    T, Bp, H = hf.shape
    N = w0.shape[-1]
    tn = _tile(N, tn_cap, 128)
    a0 = hf.reshape(T, Bp * H)
    a1 = hb.reshape(T, Bp * H)
    grid = (N // tn, Bp)
    tile_bytes = (2 * 2 * T * H * 2 + 2 * 2 * H * tn * 2 + 2 * tn * 4
                  + 2 * T * tn * 4)
    vmem_limit = int(min(64 * 1024 * 1024, max(16 * 1024 * 1024, 2 * tile_bytes)))
    return pl.pallas_call(
        _head_bm_body,
        out_shape=jax.ShapeDtypeStruct((Bp, T, N), jnp.float32),
        grid=grid,
        in_specs=[
            pl.BlockSpec((T, H), lambda n, b: (0, b)),
            pl.BlockSpec((T, H), lambda n, b: (0, b)),
            pl.BlockSpec((None, H, tn), lambda n, b: (0, 0, n)),
            pl.BlockSpec((None, H, tn), lambda n, b: (0, 0, n)),
            pl.BlockSpec((None, 1, tn), lambda n, b: (0, 0, n)),
        ],
        out_specs=pl.BlockSpec((None, T, tn), lambda n, b: (b, 0, n)),
        compiler_params=pltpu.CompilerParams(
            dimension_semantics=("parallel", "parallel"),
            vmem_limit_bytes=vmem_limit),
    )(a0, a1, w0, w1, b)


def _head_tm_body(a0_ref, a1_ref, w0_ref, w1_ref, b_ref, o_ref):
    acc = jnp.dot(a0_ref[...], w0_ref[...], preferred_element_type=jnp.float32)
    acc = acc + jnp.dot(a1_ref[...], w1_ref[...], preferred_element_type=jnp.float32)
    o_ref[...] = jnp.tanh(acc + b_ref[...])


def _head_tm(a0, a1, w0, w1, b, *, tm_cap=512, tn_cap=1024):
    M, H = a0.shape
    N = w0.shape[-1]
    tm = _tile(M, tm_cap, 8)
    tn = _tile(N, tn_cap, 128)
    grid = (1, N // tn, M // tm)
    tile_bytes = (2 * 2 * tm * H * 2 + 2 * 2 * H * tn * 2 + 2 * tn * 4
                  + 2 * tm * tn * 4)
    vmem_limit = int(min(64 * 1024 * 1024, max(16 * 1024 * 1024, 2 * tile_bytes)))
    return pl.pallas_call(
        _head_tm_body,
        out_shape=jax.ShapeDtypeStruct((1, M, N), jnp.float32),
        grid=grid,
        in_specs=[
            pl.BlockSpec((tm, H), lambda g, n, m: (m, 0)),
            pl.BlockSpec((tm, H), lambda g, n, m: (m, 0)),
            pl.BlockSpec((None, H, tn), lambda g, n, m: (g, 0, n)),
            pl.BlockSpec((None, H, tn), lambda g, n, m: (g, 0, n)),
            pl.BlockSpec((None, 1, tn), lambda g, n, m: (g, 0, n)),
        ],
        out_specs=pl.BlockSpec((None, tm, tn), lambda g, n, m: (g, m, n)),
        compiler_params=pltpu.CompilerParams(
            dimension_semantics=("parallel", "parallel", "parallel"),
            vmem_limit_bytes=vmem_limit),
    )(a0, a1, w0, w1, b)


def kernel(x, l0_fwd_wih, l0_fwd_whh, l0_fwd_b, l0_bwd_wih, l0_bwd_whh, l0_bwd_b,
           l1_fwd_wih, l1_fwd_whh, l1_fwd_b, l1_bwd_wih, l1_bwd_whh, l1_bwd_b,
           lin_w, lin_b):
    B, T, F = x.shape
    H = l0_fwd_whh.shape[0]
    N = lin_w.shape[1]
    D = N // F
    Bp = _ceil_to(B, 8)
    bf = jnp.bfloat16
    perm = functools.partial(_permute_gates, H=H)

    xt = jnp.transpose(x, (1, 0, 2))
    if Bp != B:
        xt = jnp.pad(xt, ((0, 0), (0, Bp - B), (0, 0)))

    w0 = jnp.stack([perm(l0_fwd_wih), perm(l0_bwd_wih)]).astype(bf)
    b0 = jnp.stack([perm(l0_fwd_b), perm(l0_bwd_b)])
    r0 = jnp.stack([perm(l0_fwd_whh), perm(l0_bwd_whh)]).astype(bf)
    h0f, h0b = _bilstm0(xt, w0, b0, r0)

    w1 = jnp.stack([perm(l1_fwd_wih), perm(l1_bwd_wih)]).astype(bf)
    b1 = jnp.stack([perm(l1_fwd_b), perm(l1_bwd_b)])
    r1 = jnp.stack([perm(l1_fwd_whh), perm(l1_bwd_whh)]).astype(bf)
    h1f, h1b = _bilstm1(h0f, h0b, w1, b1, r1)

    Np = _ceil_to(N, 128)
    lw, lb = lin_w, lin_b
    if Np != N:
        lw = jnp.pad(lw, ((0, 0), (0, Np - N)))
        lb = jnp.pad(lb, ((0, 0), (0, Np - N)))
    y = _head_tm(h1f.reshape(T * Bp, H), h1b.reshape(T * Bp, H),
                 lw[:H][None].astype(bf), lw[H:][None].astype(bf), lb[None])
    y = y[0][:, :N].reshape(T, Bp, N)
    y = jnp.transpose(y, (1, 0, 2))[:B].reshape(B, T * F, D)
    return y


# speedup vs baseline: 1.6655x; 1.0651x over previous
import functools

import jax
import jax.numpy as jnp
from jax.experimental import pallas as pl
from jax.experimental.pallas import tpu as pltpu


def _ceil_to(x, m):
    return (x + m - 1) // m * m


def _tile(dim, cap, align):
    if dim <= cap:
        return dim
    t = (cap // align) * align
    while t > align and dim % t:
        t -= align
    assert dim % t == 0, (dim, cap, align)
    return t


def _div_tile(dim, cap):
    for t in range(min(dim, cap), 0, -1):
        if dim % t == 0:
            return t
    return 1


def _permute_gates(w, H):
    return jnp.concatenate([w[..., :2 * H], w[..., 3 * H:], w[..., 2 * H:3 * H]],
                           axis=-1)


def _gates_body(*refs, n_in):
    a_refs = refs[:n_in]
    w_refs = refs[n_in:2 * n_in]
    b_ref = refs[2 * n_in]
    o_ref = refs[2 * n_in + 1]
    acc = jnp.dot(a_refs[0][...].astype(jnp.bfloat16), w_refs[0][...],
                  preferred_element_type=jnp.float32)
    for i in range(1, n_in):
        acc = acc + jnp.dot(a_refs[i][...].astype(jnp.bfloat16), w_refs[i][...],
                            preferred_element_type=jnp.float32)
    o_ref[...] = (acc + b_ref[...]).astype(o_ref.dtype)


def _input_gates(a_list, w_list, b, *, tm_cap=512, tn_cap=512):
    n_in = len(a_list)
    M = a_list[0].shape[0]
    G, _, N = w_list[0].shape
    tm = _tile(M, tm_cap, 8)
    tn = _tile(N, tn_cap, 128)
    grid = (G, N // tn, M // tm)
    in_specs = []
    for a in a_list:
        in_specs.append(pl.BlockSpec((tm, a.shape[1]), lambda g, n, m: (m, 0)))
    for w in w_list:
        in_specs.append(pl.BlockSpec((None, w.shape[1], tn),
                                     lambda g, n, m: (g, 0, n)))
    in_specs.append(pl.BlockSpec((None, 1, tn), lambda g, n, m: (g, 0, n)))
    tile_bytes = (sum(2 * tm * a.shape[1] * a.dtype.itemsize for a in a_list)
                  + sum(2 * w.shape[1] * tn * 2 for w in w_list)
                  + 2 * tn * 4 + 2 * tm * tn * 2)
    vmem_limit = int(min(64 * 1024 * 1024, max(16 * 1024 * 1024, 2 * tile_bytes)))
    return pl.pallas_call(
        functools.partial(_gates_body, n_in=n_in),
        out_shape=jax.ShapeDtypeStruct((G, M, N), jnp.bfloat16),
        grid=grid,
        in_specs=in_specs,
        out_specs=pl.BlockSpec((None, tm, tn), lambda g, n, m: (g, m, n)),
        compiler_params=pltpu.CompilerParams(
            dimension_semantics=("parallel", "parallel", "parallel"),
            vmem_limit_bytes=vmem_limit),
    )(*a_list, *w_list, b)


def _lstm0_body(xf_ref, xb_ref, wih_ref, bias_ref, whh_ref, hf_ref, hb_ref,
                hf_sc, cf_sc, hb_sc, cb_sc, *, H, tc):
    @pl.when(pl.program_id(1) == 0)
    def _():
        hf_sc[...] = jnp.zeros_like(hf_sc)
        cf_sc[...] = jnp.zeros_like(cf_sc)
        hb_sc[...] = jnp.zeros_like(hb_sc)
        cb_sc[...] = jnp.zeros_like(cb_sc)

    bf16 = jnp.bfloat16
    Bh = xf_ref.shape[1]
    F = xf_ref.shape[2]
    H4 = 4 * H
    gf = (jnp.dot(xf_ref[...].reshape(tc * Bh, F).astype(bf16), wih_ref[0],
                  preferred_element_type=jnp.float32)
          + bias_ref[0]).astype(bf16).reshape(tc, Bh, H4)
    gb = (jnp.dot(xb_ref[...].reshape(tc * Bh, F).astype(bf16), wih_ref[1],
                  preferred_element_type=jnp.float32)
          + bias_ref[1]).astype(bf16).reshape(tc, Bh, H4)

    wf = whh_ref[0]
    wb = whh_ref[1]
    hf, cf = hf_sc[...], cf_sc[...]
    hb, cb = hb_sc[...], cb_sc[...]
    for t in range(tc):
        tb = tc - 1 - t
        zf = gf[t].astype(jnp.float32) + jnp.dot(
            hf.astype(bf16), wf, preferred_element_type=jnp.float32)
        zb = gb[tb].astype(jnp.float32) + jnp.dot(
            hb.astype(bf16), wb, preferred_element_type=jnp.float32)
        pf = jax.nn.sigmoid(zf[:, :3 * H])
        pb = jax.nn.sigmoid(zb[:, :3 * H])
        cf = pf[:, H:2 * H] * cf + pf[:, :H] * jnp.tanh(zf[:, 3 * H:])
        cb = pb[:, H:2 * H] * cb + pb[:, :H] * jnp.tanh(zb[:, 3 * H:])
        hf = pf[:, 2 * H:] * jnp.tanh(cf)
        hb = pb[:, 2 * H:] * jnp.tanh(cb)
        hf_ref[t] = hf.astype(bf16)
        hb_ref[tb] = hb.astype(bf16)
    hf_sc[...], cf_sc[...] = hf, cf
    hb_sc[...], cb_sc[...] = hb, cb


def _bilstm0(x_tbf, wih, bias, whh, *, tc_cap=32):
    T, Bp, F = x_tbf.shape
    H4 = whh.shape[-1]
    H = H4 // 4
    tc = _div_tile(T, tc_cap)
    nc = T // tc
    nb = 2 if Bp % 16 == 0 else 1
    Bh = Bp // nb

    out_shape = [jax.ShapeDtypeStruct((T, Bp, H), jnp.bfloat16)] * 2
    return pl.pallas_call(
        functools.partial(_lstm0_body, H=H, tc=tc),
        out_shape=out_shape,
        grid=(nb, nc),
        in_specs=[
            pl.BlockSpec((tc, Bh, F), lambda b, c: (c, b, 0)),
            pl.BlockSpec((tc, Bh, F), lambda b, c, nc=nc: (nc - 1 - c, b, 0)),
            pl.BlockSpec((2, F, H4), lambda b, c: (0, 0, 0)),
            pl.BlockSpec((2, 1, H4), lambda b, c: (0, 0, 0)),
            pl.BlockSpec((2, H, H4), lambda b, c: (0, 0, 0)),
        ],
        out_specs=[
            pl.BlockSpec((tc, Bh, H), lambda b, c: (c, b, 0)),
            pl.BlockSpec((tc, Bh, H), lambda b, c, nc=nc: (nc - 1 - c, b, 0)),
        ],
        scratch_shapes=[
            pltpu.VMEM((Bh, H), jnp.float32),
            pltpu.VMEM((Bh, H), jnp.float32),
            pltpu.VMEM((Bh, H), jnp.float32),
            pltpu.VMEM((Bh, H), jnp.float32),
        ],
        compiler_params=pltpu.CompilerParams(
            dimension_semantics=("parallel", "arbitrary")),
    )(x_tbf, x_tbf, wih, bias, whh)


def _lstm1_body(af_ref, bf_ref, ab_ref, bb_ref, wih_ref, bias_ref, whh_ref,
                hf_ref, hb_ref, hf_sc, cf_sc, hb_sc, cb_sc, *, H, tc):
    @pl.when(pl.program_id(1) == 0)
    def _():
        hf_sc[...] = jnp.zeros_like(hf_sc)
        cf_sc[...] = jnp.zeros_like(cf_sc)
        hb_sc[...] = jnp.zeros_like(hb_sc)
        cb_sc[...] = jnp.zeros_like(cb_sc)

    bf16 = jnp.bfloat16
    Bh = af_ref.shape[1]
    H4 = 4 * H
    gf = (jnp.dot(af_ref[...].reshape(tc * Bh, H), wih_ref[0, :H],
                  preferred_element_type=jnp.float32)
          + jnp.dot(bf_ref[...].reshape(tc * Bh, H), wih_ref[0, H:],
                    preferred_element_type=jnp.float32)
          + bias_ref[0]).astype(bf16).reshape(tc, Bh, H4)
    gb = (jnp.dot(ab_ref[...].reshape(tc * Bh, H), wih_ref[1, :H],
                  preferred_element_type=jnp.float32)
          + jnp.dot(bb_ref[...].reshape(tc * Bh, H), wih_ref[1, H:],
                    preferred_element_type=jnp.float32)
          + bias_ref[1]).astype(bf16).reshape(tc, Bh, H4)

    wf = whh_ref[0]
    wb = whh_ref[1]
    hf, cf = hf_sc[...], cf_sc[...]
    hb, cb = hb_sc[...], cb_sc[...]
    for t in range(tc):
        tb = tc - 1 - t
        zf = gf[t].astype(jnp.float32) + jnp.dot(
            hf.astype(bf16), wf, preferred_element_type=jnp.float32)
        zb = gb[tb].astype(jnp.float32) + jnp.dot(
            hb.astype(bf16), wb, preferred_element_type=jnp.float32)
        pf = jax.nn.sigmoid(zf[:, :3 * H])
        pb = jax.nn.sigmoid(zb[:, :3 * H])
        cf = pf[:, H:2 * H] * cf + pf[:, :H] * jnp.tanh(zf[:, 3 * H:])
        cb = pb[:, H:2 * H] * cb + pb[:, :H] * jnp.tanh(zb[:, 3 * H:])
        hf = pf[:, 2 * H:] * jnp.tanh(cf)
        hb = pb[:, 2 * H:] * jnp.tanh(cb)
        hf_ref[t] = hf.astype(bf16)
        hb_ref[tb] = hb.astype(bf16)
    hf_sc[...], cf_sc[...] = hf, cf
    hb_sc[...], cb_sc[...] = hb, cb


def _bilstm1(h0f, h0b, wih, bias, whh, *, tc_cap=32):
    T, Bp, H = h0f.shape
    H4 = whh.shape[-1]
    tc = _div_tile(T, tc_cap)
    nc = T // tc
    nb = 2 if Bp % 16 == 0 else 1
    Bh = Bp // nb

    fwd = lambda b, c: (c, b, 0)
    bwd = lambda b, c, nc=nc: (nc - 1 - c, b, 0)
    out_shape = [jax.ShapeDtypeStruct((T, Bp, H), jnp.bfloat16)] * 2
    return pl.pallas_call(
        functools.partial(_lstm1_body, H=H, tc=tc),
        out_shape=out_shape,
        grid=(nb, nc),
        in_specs=[
            pl.BlockSpec((tc, Bh, H), fwd),
            pl.BlockSpec((tc, Bh, H), fwd),
            pl.BlockSpec((tc, Bh, H), bwd),
            pl.BlockSpec((tc, Bh, H), bwd),
            pl.BlockSpec((2, 2 * H, H4), lambda b, c: (0, 0, 0)),
            pl.BlockSpec((2, 1, H4), lambda b, c: (0, 0, 0)),
            pl.BlockSpec((2, H, H4), lambda b, c: (0, 0, 0)),
        ],
        out_specs=[
            pl.BlockSpec((tc, Bh, H), fwd),
            pl.BlockSpec((tc, Bh, H), bwd),
        ],
        scratch_shapes=[
            pltpu.VMEM((Bh, H), jnp.float32),
            pltpu.VMEM((Bh, H), jnp.float32),
            pltpu.VMEM((Bh, H), jnp.float32),
            pltpu.VMEM((Bh, H), jnp.float32),
        ],
        compiler_params=pltpu.CompilerParams(
            dimension_semantics=("parallel", "arbitrary")),
    )(h0f, h0b, h0f, h0b, wih, bias, whh)


def _lstm_body(gf_ref, gb_ref, whh_ref, hf_ref, hb_ref,
               hf_sc, cf_sc, hb_sc, cb_sc, *, H, tc):
    @pl.when(pl.program_id(1) == 0)
    def _():
        hf_sc[...] = jnp.zeros_like(hf_sc)
        cf_sc[...] = jnp.zeros_like(cf_sc)
        hb_sc[...] = jnp.zeros_like(hb_sc)
        cb_sc[...] = jnp.zeros_like(cb_sc)

    wf = whh_ref[0]
    wb = whh_ref[1]
    hf, cf = hf_sc[...], cf_sc[...]
    hb, cb = hb_sc[...], cb_sc[...]
    bf16 = jnp.bfloat16
    for t in range(tc):
        tb = tc - 1 - t
        zf = gf_ref[t].astype(jnp.float32) + jnp.dot(
            hf.astype(bf16), wf, preferred_element_type=jnp.float32)
        zb = gb_ref[tb].astype(jnp.float32) + jnp.dot(
            hb.astype(bf16), wb, preferred_element_type=jnp.float32)
        pf = jax.nn.sigmoid(zf[:, :3 * H])
        pb = jax.nn.sigmoid(zb[:, :3 * H])
        cf = pf[:, H:2 * H] * cf + pf[:, :H] * jnp.tanh(zf[:, 3 * H:])
        cb = pb[:, H:2 * H] * cb + pb[:, :H] * jnp.tanh(zb[:, 3 * H:])
        hf = pf[:, 2 * H:] * jnp.tanh(cf)
        hb = pb[:, 2 * H:] * jnp.tanh(cb)
        hf_ref[t] = hf.astype(bf16)
        hb_ref[tb] = hb.astype(bf16)
    hf_sc[...], cf_sc[...] = hf, cf
    hb_sc[...], cb_sc[...] = hb, cb


def _bilstm(g, whh, *, tc_cap=32):
    _, T, Bp, H4 = g.shape
    H = H4 // 4
    tc = _div_tile(T, tc_cap)
    nc = T // tc
    nb = 2 if Bp % 16 == 0 else 1
    Bh = Bp // nb

    out_shape = [jax.ShapeDtypeStruct((T, Bp, H), jnp.bfloat16)] * 2
    return pl.pallas_call(
        functools.partial(_lstm_body, H=H, tc=tc),
        out_shape=out_shape,
        grid=(nb, nc),
        in_specs=[
            pl.BlockSpec((None, tc, Bh, H4), lambda b, c: (0, c, b, 0)),
            pl.BlockSpec((None, tc, Bh, H4),
                         lambda b, c, nc=nc: (1, nc - 1 - c, b, 0)),
            pl.BlockSpec((2, H, H4), lambda b, c: (0, 0, 0)),
        ],
        out_specs=[
            pl.BlockSpec((tc, Bh, H), lambda b, c: (c, b, 0)),
            pl.BlockSpec((tc, Bh, H), lambda b, c, nc=nc: (nc - 1 - c, b, 0)),
        ],
        scratch_shapes=[
            pltpu.VMEM((Bh, H), jnp.float32),
            pltpu.VMEM((Bh, H), jnp.float32),
            pltpu.VMEM((Bh, H), jnp.float32),
            pltpu.VMEM((Bh, H), jnp.float32),
        ],
        compiler_params=pltpu.CompilerParams(
            dimension_semantics=("parallel", "arbitrary")),
    )(g, g, whh)


def _head_body(h_ref, w_ref, b_ref, o_ref, *, H, tt):
    a0 = h_ref[0].reshape(-1, H)
    a1 = h_ref[1].reshape(-1, H)
    acc = jnp.dot(a0, w_ref[0], preferred_element_type=jnp.float32)
    acc = acc + jnp.dot(a1, w_ref[1], preferred_element_type=jnp.float32)
    y = jnp.tanh(acc + b_ref[...])
    y = y.reshape(tt, -1, y.shape[-1])
    for i in range(tt):
        o_ref[:, i, :] = y[i]


def _head(h, w, b, *, tt_cap=8, tn_cap=512):
    _, T, Bp, H = h.shape
    N = w.shape[-1]
    tn = _tile(N, tn_cap, 128)
    tt = _div_tile(T, tt_cap)
    grid = (N // tn, T // tt)
    return pl.pallas_call(
        functools.partial(_head_body, H=H, tt=tt),
        out_shape=jax.ShapeDtypeStruct((Bp, T, N), jnp.float32),
        grid=grid,
        in_specs=[
            pl.BlockSpec((2, tt, Bp, H), lambda n, t: (0, t, 0, 0)),
            pl.BlockSpec((2, H, tn), lambda n, t: (0, 0, n)),
            pl.BlockSpec((1, tn), lambda n, t: (0, n)),
        ],
        out_specs=pl.BlockSpec((Bp, tt, tn), lambda n, t: (0, t, n)),
        compiler_params=pltpu.CompilerParams(
            dimension_semantics=("parallel", "parallel")),
    )(h, w, b)


def _head_bm_body(a0_ref, a1_ref, w0_ref, w1_ref, b_ref, o_ref):
    acc = jnp.dot(a0_ref[...], w0_ref[...], preferred_element_type=jnp.float32)
    acc = acc + jnp.dot(a1_ref[...], w1_ref[...], preferred_element_type=jnp.float32)
    o_ref[...] = jnp.tanh(acc + b_ref[...])


def _head_bm(hf, hb, w0, w1, b, *, tn_cap=512):
    T, Bp, H = hf.shape
    N = w0.shape[-1]
    tn = _tile(N, tn_cap, 128)
    a0 = hf.reshape(T, Bp * H)
    a1 = hb.reshape(T, Bp * H)
    grid = (N // tn, Bp)
    tile_bytes = (2 * 2 * T * H * 2 + 2 * 2 * H * tn * 2 + 2 * tn * 4
                  + 2 * T * tn * 4)
    vmem_limit = int(min(64 * 1024 * 1024, max(16 * 1024 * 1024, 2 * tile_bytes)))
    return pl.pallas_call(
        _head_bm_body,
        out_shape=jax.ShapeDtypeStruct((Bp, T, N), jnp.float32),
        grid=grid,
        in_specs=[
            pl.BlockSpec((T, H), lambda n, b: (0, b)),
            pl.BlockSpec((T, H), lambda n, b: (0, b)),
            pl.BlockSpec((None, H, tn), lambda n, b: (0, 0, n)),
            pl.BlockSpec((None, H, tn), lambda n, b: (0, 0, n)),
            pl.BlockSpec((None, 1, tn), lambda n, b: (0, 0, n)),
        ],
        out_specs=pl.BlockSpec((None, T, tn), lambda n, b: (b, 0, n)),
        compiler_params=pltpu.CompilerParams(
            dimension_semantics=("parallel", "parallel"),
            vmem_limit_bytes=vmem_limit),
    )(a0, a1, w0, w1, b)


def _head_tm_body(a0_ref, a1_ref, w0_ref, w1_ref, b_ref, o_ref):
    acc = jnp.dot(a0_ref[...], w0_ref[...], preferred_element_type=jnp.float32)
    acc = acc + jnp.dot(a1_ref[...], w1_ref[...], preferred_element_type=jnp.float32)
    o_ref[...] = jnp.tanh(acc + b_ref[...])


def _head_tm(a0, a1, w0, w1, b, *, tm_cap=512, tn_cap=5120):
    M, H = a0.shape
    N = w0.shape[-1]
    tm = _tile(M, tm_cap, 8)
    tn = _tile(N, tn_cap, 128)
    grid = (1, N // tn, M // tm)
    tile_bytes = (2 * 2 * tm * H * 2 + 2 * 2 * H * tn * 2 + 2 * tn * 4
                  + 2 * tm * tn * 4)
    vmem_limit = int(min(64 * 1024 * 1024, max(16 * 1024 * 1024, 2 * tile_bytes)))
    return pl.pallas_call(
        _head_tm_body,
        out_shape=jax.ShapeDtypeStruct((1, M, N), jnp.float32),
        grid=grid,
        in_specs=[
            pl.BlockSpec((tm, H), lambda g, n, m: (m, 0)),
            pl.BlockSpec((tm, H), lambda g, n, m: (m, 0)),
            pl.BlockSpec((None, H, tn), lambda g, n, m: (g, 0, n)),
            pl.BlockSpec((None, H, tn), lambda g, n, m: (g, 0, n)),
            pl.BlockSpec((None, 1, tn), lambda g, n, m: (g, 0, n)),
        ],
        out_specs=pl.BlockSpec((None, tm, tn), lambda g, n, m: (g, m, n)),
        compiler_params=pltpu.CompilerParams(
            dimension_semantics=("parallel", "parallel", "parallel"),
            vmem_limit_bytes=vmem_limit),
    )(a0, a1, w0, w1, b)


def kernel(x, l0_fwd_wih, l0_fwd_whh, l0_fwd_b, l0_bwd_wih, l0_bwd_whh, l0_bwd_b,
           l1_fwd_wih, l1_fwd_whh, l1_fwd_b, l1_bwd_wih, l1_bwd_whh, l1_bwd_b,
           lin_w, lin_b):
    B, T, F = x.shape
    H = l0_fwd_whh.shape[0]
    N = lin_w.shape[1]
    D = N // F
    Bp = _ceil_to(B, 8)
    bf = jnp.bfloat16
    perm = functools.partial(_permute_gates, H=H)

    xt = jnp.transpose(x, (1, 0, 2))
    if Bp != B:
        xt = jnp.pad(xt, ((0, 0), (0, Bp - B), (0, 0)))

    w0 = jnp.stack([perm(l0_fwd_wih), perm(l0_bwd_wih)]).astype(bf)
    b0 = jnp.stack([perm(l0_fwd_b), perm(l0_bwd_b)])
    r0 = jnp.stack([perm(l0_fwd_whh), perm(l0_bwd_whh)]).astype(bf)
    h0f, h0b = _bilstm0(xt, w0, b0, r0)

    w1 = jnp.stack([perm(l1_fwd_wih), perm(l1_bwd_wih)]).astype(bf)
    b1 = jnp.stack([perm(l1_fwd_b), perm(l1_bwd_b)])
    r1 = jnp.stack([perm(l1_fwd_whh), perm(l1_bwd_whh)]).astype(bf)
    h1f, h1b = _bilstm1(h0f, h0b, w1, b1, r1)

    Np = _ceil_to(N, 128)
    lw, lb = lin_w, lin_b
    if Np != N:
        lw = jnp.pad(lw, ((0, 0), (0, Np - N)))
        lb = jnp.pad(lb, ((0, 0), (0, Np - N)))
    y = _head_tm(h1f.reshape(T * Bp, H), h1b.reshape(T * Bp, H),
                 lw[:H][None].astype(bf), lw[H:][None].astype(bf), lb[None])
    y = y[0][:, :N].reshape(T, Bp, N)
    y = jnp.transpose(y, (1, 0, 2))[:B].reshape(B, T * F, D)
    return y


# speedup vs baseline: 1.6659x; 1.0002x over previous
import functools

import jax
import jax.numpy as jnp
from jax.experimental import pallas as pl
from jax.experimental.pallas import tpu as pltpu


def _ceil_to(x, m):
    return (x + m - 1) // m * m


def _tile(dim, cap, align):
    if dim <= cap:
        return dim
    t = (cap // align) * align
    while t > align and dim % t:
        t -= align
    assert dim % t == 0, (dim, cap, align)
    return t


def _div_tile(dim, cap):
    for t in range(min(dim, cap), 0, -1):
        if dim % t == 0:
            return t
    return 1


def _permute_gates(w, H):
    return jnp.concatenate([w[..., :2 * H], w[..., 3 * H:], w[..., 2 * H:3 * H]],
                           axis=-1)


def _lstm_steps(gf, gb, whh_ref, hf_ref, hb_ref,
                hf_sc, cf_sc, hb_sc, cb_sc, *, H, tc):
    bf16 = jnp.bfloat16
    wf = whh_ref[0]
    wb = whh_ref[1]
    hf, cf = hf_sc[...], cf_sc[...]
    hb, cb = hb_sc[...], cb_sc[...]
    for t in range(tc):
        tb = tc - 1 - t
        zf = gf[t].astype(jnp.float32) + jnp.dot(
            hf.astype(bf16), wf, preferred_element_type=jnp.float32)
        zb = gb[tb].astype(jnp.float32) + jnp.dot(
            hb.astype(bf16), wb, preferred_element_type=jnp.float32)
        pf = jax.nn.sigmoid(zf[:, :3 * H])
        pb = jax.nn.sigmoid(zb[:, :3 * H])
        cf = pf[:, H:2 * H] * cf + pf[:, :H] * jnp.tanh(zf[:, 3 * H:])
        cb = pb[:, H:2 * H] * cb + pb[:, :H] * jnp.tanh(zb[:, 3 * H:])
        hf = pf[:, 2 * H:] * jnp.tanh(cf)
        hb = pb[:, 2 * H:] * jnp.tanh(cb)
        hf_ref[t] = hf.astype(bf16)
        hb_ref[tb] = hb.astype(bf16)
    hf_sc[...], cf_sc[...] = hf, cf
    hb_sc[...], cb_sc[...] = hb, cb


def _lstm0_body(xf_ref, xb_ref, wih_ref, bias_ref, whh_ref, hf_ref, hb_ref,
                hf_sc, cf_sc, hb_sc, cb_sc, *, H, tc):
    @pl.when(pl.program_id(1) == 0)
    def _():
        hf_sc[...] = jnp.zeros_like(hf_sc)
        cf_sc[...] = jnp.zeros_like(cf_sc)
        hb_sc[...] = jnp.zeros_like(hb_sc)
        cb_sc[...] = jnp.zeros_like(cb_sc)

    bf16 = jnp.bfloat16
    Bh, F = xf_ref.shape[1], xf_ref.shape[2]
    H4 = 4 * H
    gf = (jnp.dot(xf_ref[...].reshape(tc * Bh, F).astype(bf16), wih_ref[0],
                  preferred_element_type=jnp.float32)
          + bias_ref[0]).astype(bf16).reshape(tc, Bh, H4)
    gb = (jnp.dot(xb_ref[...].reshape(tc * Bh, F).astype(bf16), wih_ref[1],
                  preferred_element_type=jnp.float32)
          + bias_ref[1]).astype(bf16).reshape(tc, Bh, H4)
    _lstm_steps(gf, gb, whh_ref, hf_ref, hb_ref,
                hf_sc, cf_sc, hb_sc, cb_sc, H=H, tc=tc)


def _bilstm0(x_tbf, wih, bias, whh, *, tc_cap=32):
    T, Bp, F = x_tbf.shape
    H4 = whh.shape[-1]
    H = H4 // 4
    tc = _div_tile(T, tc_cap)
    nc = T // tc
    nb = 2 if Bp % 16 == 0 else 1
    Bh = Bp // nb

    fwd = lambda b, c: (c, b, 0)
    bwd = lambda b, c, nc=nc: (nc - 1 - c, b, 0)
    out_shape = [jax.ShapeDtypeStruct((T, Bp, H), jnp.bfloat16)] * 2
    return pl.pallas_call(
        functools.partial(_lstm0_body, H=H, tc=tc),
        out_shape=out_shape,
        grid=(nb, nc),
        in_specs=[
            pl.BlockSpec((tc, Bh, F), fwd),
            pl.BlockSpec((tc, Bh, F), bwd),
            pl.BlockSpec((2, F, H4), lambda b, c: (0, 0, 0)),
            pl.BlockSpec((2, 1, H4), lambda b, c: (0, 0, 0)),
            pl.BlockSpec((2, H, H4), lambda b, c: (0, 0, 0)),
        ],
        out_specs=[
            pl.BlockSpec((tc, Bh, H), fwd),
            pl.BlockSpec((tc, Bh, H), bwd),
        ],
        scratch_shapes=[
            pltpu.VMEM((Bh, H), jnp.float32),
            pltpu.VMEM((Bh, H), jnp.float32),
            pltpu.VMEM((Bh, H), jnp.float32),
            pltpu.VMEM((Bh, H), jnp.float32),
        ],
        compiler_params=pltpu.CompilerParams(
            dimension_semantics=("parallel", "arbitrary")),
    )(x_tbf, x_tbf, wih, bias, whh)


def _lstm1_body(af_ref, bf_ref, ab_ref, bb_ref, wih_ref, bias_ref, whh_ref,
                hf_ref, hb_ref, hf_sc, cf_sc, hb_sc, cb_sc, *, H, tc):
    @pl.when(pl.program_id(1) == 0)
    def _():
        hf_sc[...] = jnp.zeros_like(hf_sc)
        cf_sc[...] = jnp.zeros_like(cf_sc)
        hb_sc[...] = jnp.zeros_like(hb_sc)
        cb_sc[...] = jnp.zeros_like(cb_sc)

    bf16 = jnp.bfloat16
    Bh = af_ref.shape[1]
    H4 = 4 * H
    gf = (jnp.dot(af_ref[...].reshape(tc * Bh, H), wih_ref[0, :H],
                  preferred_element_type=jnp.float32)
          + jnp.dot(bf_ref[...].reshape(tc * Bh, H), wih_ref[0, H:],
                    preferred_element_type=jnp.float32)
          + bias_ref[0]).astype(bf16).reshape(tc, Bh, H4)
    gb = (jnp.dot(ab_ref[...].reshape(tc * Bh, H), wih_ref[1, :H],
                  preferred_element_type=jnp.float32)
          + jnp.dot(bb_ref[...].reshape(tc * Bh, H), wih_ref[1, H:],
                    preferred_element_type=jnp.float32)
          + bias_ref[1]).astype(bf16).reshape(tc, Bh, H4)
    _lstm_steps(gf, gb, whh_ref, hf_ref, hb_ref,
                hf_sc, cf_sc, hb_sc, cb_sc, H=H, tc=tc)


def _bilstm1(h0f, h0b, wih, bias, whh, *, tc_cap=32):
    T, Bp, H = h0f.shape
    H4 = whh.shape[-1]
    tc = _div_tile(T, tc_cap)
    nc = T // tc
    nb = 2 if Bp % 16 == 0 else 1
    Bh = Bp // nb

    fwd = lambda b, c: (c, b, 0)
    bwd = lambda b, c, nc=nc: (nc - 1 - c, b, 0)
    out_shape = [jax.ShapeDtypeStruct((T, Bp, H), jnp.bfloat16)] * 2
    return pl.pallas_call(
        functools.partial(_lstm1_body, H=H, tc=tc),
        out_shape=out_shape,
        grid=(nb, nc),
        in_specs=[
            pl.BlockSpec((tc, Bh, H), fwd),
            pl.BlockSpec((tc, Bh, H), fwd),
            pl.BlockSpec((tc, Bh, H), bwd),
            pl.BlockSpec((tc, Bh, H), bwd),
            pl.BlockSpec((2, 2 * H, H4), lambda b, c: (0, 0, 0)),
            pl.BlockSpec((2, 1, H4), lambda b, c: (0, 0, 0)),
            pl.BlockSpec((2, H, H4), lambda b, c: (0, 0, 0)),
        ],
        out_specs=[
            pl.BlockSpec((tc, Bh, H), fwd),
            pl.BlockSpec((tc, Bh, H), bwd),
        ],
        scratch_shapes=[
            pltpu.VMEM((Bh, H), jnp.float32),
            pltpu.VMEM((Bh, H), jnp.float32),
            pltpu.VMEM((Bh, H), jnp.float32),
            pltpu.VMEM((Bh, H), jnp.float32),
        ],
        compiler_params=pltpu.CompilerParams(
            dimension_semantics=("parallel", "arbitrary")),
    )(h0f, h0b, h0f, h0b, wih, bias, whh)


def _head_body(a0_ref, a1_ref, w0_ref, w1_ref, b_ref, o_ref):
    acc = jnp.dot(a0_ref[...], w0_ref[...], preferred_element_type=jnp.float32)
    acc = acc + jnp.dot(a1_ref[...], w1_ref[...], preferred_element_type=jnp.float32)
    o_ref[...] = jnp.tanh(acc + b_ref[...])


def _head(a0, a1, w0, w1, b, *, tm_cap=512, tn_cap=5120):
    M, H = a0.shape
    N = w0.shape[-1]
    tm = _tile(M, tm_cap, 8)
    tn = _tile(N, tn_cap, 128)
    grid = (1, N // tn, M // tm)
    tile_bytes = (2 * 2 * tm * H * 2 + 2 * 2 * H * tn * 2 + 2 * tn * 4
                  + 2 * tm * tn * 4)
    vmem_limit = int(min(64 * 1024 * 1024, max(16 * 1024 * 1024, 2 * tile_bytes)))
    return pl.pallas_call(
        _head_body,
        out_shape=jax.ShapeDtypeStruct((1, M, N), jnp.float32),
        grid=grid,
        in_specs=[
            pl.BlockSpec((tm, H), lambda g, n, m: (m, 0)),
            pl.BlockSpec((tm, H), lambda g, n, m: (m, 0)),
            pl.BlockSpec((None, H, tn), lambda g, n, m: (g, 0, n)),
            pl.BlockSpec((None, H, tn), lambda g, n, m: (g, 0, n)),
            pl.BlockSpec((None, 1, tn), lambda g, n, m: (g, 0, n)),
        ],
        out_specs=pl.BlockSpec((None, tm, tn), lambda g, n, m: (g, m, n)),
        compiler_params=pltpu.CompilerParams(
            dimension_semantics=("parallel", "parallel", "parallel"),
            vmem_limit_bytes=vmem_limit),
    )(a0, a1, w0, w1, b)


def kernel(x, l0_fwd_wih, l0_fwd_whh, l0_fwd_b, l0_bwd_wih, l0_bwd_whh, l0_bwd_b,
           l1_fwd_wih, l1_fwd_whh, l1_fwd_b, l1_bwd_wih, l1_bwd_whh, l1_bwd_b,
           lin_w, lin_b):
    B, T, F = x.shape
    H = l0_fwd_whh.shape[0]
    N = lin_w.shape[1]
    D = N // F
    Bp = _ceil_to(B, 8)
    bf = jnp.bfloat16
    perm = functools.partial(_permute_gates, H=H)

    xt = jnp.transpose(x, (1, 0, 2))
    if Bp != B:
        xt = jnp.pad(xt, ((0, 0), (0, Bp - B), (0, 0)))

    w0 = jnp.stack([perm(l0_fwd_wih), perm(l0_bwd_wih)]).astype(bf)
    b0 = jnp.stack([perm(l0_fwd_b), perm(l0_bwd_b)])
    r0 = jnp.stack([perm(l0_fwd_whh), perm(l0_bwd_whh)]).astype(bf)
    h0f, h0b = _bilstm0(xt, w0, b0, r0)

    w1 = jnp.stack([perm(l1_fwd_wih), perm(l1_bwd_wih)]).astype(bf)
    b1 = jnp.stack([perm(l1_fwd_b), perm(l1_bwd_b)])
    r1 = jnp.stack([perm(l1_fwd_whh), perm(l1_bwd_whh)]).astype(bf)
    h1f, h1b = _bilstm1(h0f, h0b, w1, b1, r1)

    Np = _ceil_to(N, 128)
    lw, lb = lin_w, lin_b
    if Np != N:
        lw = jnp.pad(lw, ((0, 0), (0, Np - N)))
        lb = jnp.pad(lb, ((0, 0), (0, Np - N)))
    y = _head(h1f.reshape(T * Bp, H), h1b.reshape(T * Bp, H),
              lw[:H][None].astype(bf), lw[H:][None].astype(bf), lb[None])
    y = y[0][:, :N].reshape(T, Bp, N)
    y = jnp.transpose(y, (1, 0, 2))[:B].reshape(B, T * F, D)
    return y
